# baseline (speedup 1.0000x reference)
import functools

import jax
import jax.numpy as jnp
from jax.experimental import pallas as pl
from jax.experimental.pallas import tpu as pltpu
from jax.experimental.pallas import tpu_sc as plsc

B, S, D, H = 2, 2048, 768, 12
FFN, INTER, E, ND = 3072, 768, 8, 4
DH = D // H
T = B * S
SCALE = DH ** -0.5

TB_QKV = 512
BQ = 512
TB = 256
NTB = T // TB

TBS = 256
NB = 24
PAD_T = NB * TBS

TROWS = T // 128


def _gelu(x):
    return x * 0.5 * (1.0 + jax.lax.erf(x * (2.0 ** -0.5)))


def _layernorm(t, g, b):
    m = jnp.mean(t, axis=-1, keepdims=True)
    v = jnp.mean((t - m) ** 2, axis=-1, keepdims=True)
    return (t - m) / jnp.sqrt(v + 1e-5) * g + b


def _qkv_body(x_ref, w_ref, b_ref, o_ref):
    o_ref[...] = (jnp.dot(x_ref[...], w_ref[...],
                          preferred_element_type=jnp.float32) + b_ref[...])


def _attn_body(q_ref, k_ref, v_ref, o_ref):
    q = q_ref[0, 0]
    k = k_ref[0, 0]
    v = v_ref[0, 0]
    s = jax.lax.dot_general(q, k, (((1,), (1,)), ((), ())),
                            preferred_element_type=jnp.float32)
    m = jnp.max(s, axis=-1, keepdims=True)
    p = jnp.exp(s - m)
    p = p / jnp.sum(p, axis=-1, keepdims=True)
    o_ref[0, 0] = jnp.dot(p, v, preferred_element_type=jnp.float32)


def _oproj_ln_body(o_ref, w_ref, b_ref, hs_ref, g_ref, bb_ref, x_ref):
    t = (jnp.dot(o_ref[...], w_ref[...], preferred_element_type=jnp.float32)
         + b_ref[...] + hs_ref[...])
    x_ref[...] = _layernorm(t, g_ref[...], bb_ref[...])


def _gate_body(idx_ref, x_ref, gw_ref, gb_ref, gate_ref, gval_ref):
    del idx_ref
    x = x_ref[...]
    gw = gw_ref[0]
    logits = jax.lax.dot_general(x, gw, (((1,), (1,)), ((), ())),
                                 preferred_element_type=jnp.float32)
    logits = logits + gb_ref[0]
    m = jnp.max(logits, axis=-1, keepdims=True)
    p = jnp.exp(logits - m)
    sp = jnp.sum(p, axis=-1)
    gate_ref[0, 0] = jnp.argmax(logits, axis=-1).astype(jnp.int32)
    gval_ref[0, 0] = jnp.max(p, axis=-1) / sp


def _route_body(gate_ref, dst_ref, bex_ref, src_ref, dsts_ref):
    j = pl.program_id(0)

    @pl.when(j == 0)
    def _meta():
        g = gate_ref[...]
        l0 = jax.lax.broadcasted_iota(jnp.int32, (128, 128), 0)
        l1 = jax.lax.broadcasted_iota(jnp.int32, (128, 128), 1)
        slt_lane = (l0 < l1).astype(jnp.float32)
        r0 = jax.lax.broadcasted_iota(jnp.int32, (TROWS, TROWS), 0)
        r1 = jax.lax.broadcasted_iota(jnp.int32, (TROWS, TROWS), 1)
        slt_row = (r1 < r0).astype(jnp.float32)

        dst = jnp.zeros((TROWS, 128), jnp.float32)
        bpos = (jax.lax.broadcasted_iota(jnp.int32, (1, NB), 1)
                * TBS).astype(jnp.float32)
        bex = jnp.zeros((1, NB), jnp.float32)
        base = jnp.float32(0.0)
        for e in range(E):
            oh = (g == e).astype(jnp.float32)
            cs = jnp.dot(oh, slt_lane,
                         preferred_element_type=jnp.float32)
            rs = jnp.sum(oh, axis=1, keepdims=True)
            rp = jnp.dot(slt_row, rs,
                         preferred_element_type=jnp.float32)
            dst = dst + oh * (cs + rp + base)
            cnt = jnp.sum(oh)
            pc = jnp.ceil(cnt * (1.0 / TBS)) * TBS
            bex = bex + jnp.float32(e) * ((bpos >= base) &
                                          (bpos < base + pc)).astype(jnp.float32)
            base = base + pc
        dst_ref[...] = dst.astype(jnp.int32)
        dsts_ref[...] = dst.astype(jnp.int32)
        bex_ref[...] = bex.astype(jnp.int32)
        src_ref[0, 0] = jnp.zeros((TBS,), jnp.int32)

    @pl.when(j > 0)
    def _src():
        p0 = (j - 1) * TBS
        d = dsts_ref[...]
        pos = (jax.lax.broadcasted_iota(jnp.int32, (TBS, TROWS, 128), 0)
               + p0)
        eq = (d[None, :, :] == pos).astype(jnp.float32)
        tok = (jax.lax.broadcasted_iota(jnp.int32, (TBS, TROWS, 128), 1)
               * 128
               + jax.lax.broadcasted_iota(jnp.int32, (TBS, TROWS, 128), 2)
               ).astype(jnp.float32)
        s = jnp.sum(jnp.sum(eq * tok, axis=2), axis=1)
        src_ref[0, 0] = s.astype(jnp.int32)


def _expert_body(bex_ref, xs_ref, e1w_ref, e1b_ref, e2w_ref, ys_ref):
    del bex_ref
    xs = xs_ref[...]
    he = jax.lax.dot_general(xs, e1w_ref[0], (((1,), (1,)), ((), ())),
                             preferred_element_type=jnp.float32)
    he = _gelu(he + e1b_ref[0])
    ys_ref[...] = jax.lax.dot_general(he, e2w_ref[0], (((1,), (1,)), ((), ())),
                                      preferred_element_type=jnp.float32)


def _shared_body(x_ref, w1_ref, b1_ref, w2_ref, o_ref):
    h = jnp.dot(x_ref[...], w1_ref[...], preferred_element_type=jnp.float32)
    h = _gelu(h + b1_ref[...])
    o_ref[...] = jnp.dot(h, w2_ref[...], preferred_element_type=jnp.float32)


def _combine_body(ysh_ref, ye_ref, b2_ref, gval_ref, x_ref, g_ref, bb_ref,
                  o_ref):
    x = x_ref[...]
    y = ysh_ref[...] + ye_ref[...] + b2_ref[...]
    t = y * gval_ref[0, 0][:, None] + x
    o_ref[...] = _layernorm(t, g_ref[...], bb_ref[...])


def _sc_gather(table, idx, n_rows):
    info = plsc.get_sparse_core_info()
    nc, ns = info.num_cores, info.num_subcores
    nw = nc * ns
    per_w = n_rows // nw
    ch = 64
    n_ch = per_w // ch
    mesh = plsc.VectorSubcoreMesh(core_axis_name="c", subcore_axis_name="s")

    @functools.partial(
        pl.kernel, mesh=mesh,
        out_type=jax.ShapeDtypeStruct((n_rows, D), jnp.float32),
        scratch_types=[pltpu.VMEM((per_w,), jnp.int32),
                       pltpu.VMEM((ch, D), jnp.float32),
                       pltpu.VMEM((ch, D), jnp.float32),
                       pltpu.SemaphoreType.DMA,
                       pltpu.SemaphoreType.DMA])
    def gk(table_hbm, idx_hbm, out_hbm, idx_v, b0, b1, gsem, ssem):
        bufs = [b0, b1]
        wid = jax.lax.axis_index("s") * nc + jax.lax.axis_index("c")
        base = wid * per_w
        pltpu.sync_copy(idx_hbm.at[pl.ds(base, per_w)], idx_v)
        gathers = [pltpu.async_copy(table_hbm.at[idx_v.at[pl.ds(0, ch)]],
                                    b0, gsem)]
        stores = []
        for c in range(n_ch):
            if c + 1 < n_ch:
                if c >= 1:
                    stores[c - 1].wait()
                gathers.append(pltpu.async_copy(
                    table_hbm.at[idx_v.at[pl.ds((c + 1) * ch, ch)]],
                    bufs[(c + 1) % 2], gsem))
            gathers[c].wait()
            stores.append(pltpu.async_copy(
                bufs[c % 2], out_hbm.at[pl.ds(base + c * ch, ch)], ssem))
        stores[-1].wait()
        if n_ch >= 2:
            stores[-2].wait()

    return gk(table, idx)


def kernel(hidden_states, idxes, Wq, bq, Wk, bk, Wv, bv, Wo, bo, ln1_g, ln1_b,
           fc1_W, fc1_b, fc2_W, fc2_b, exp1_W, exp1_b, exp2_W, gate_W, gate_b,
           fln_g, fln_b):
    xf = hidden_states.reshape(T, D)
    Wqkv = jnp.concatenate([Wq.T * SCALE, Wk.T, Wv.T], axis=1)
    bqkv = jnp.concatenate([bq * SCALE, bk, bv]).reshape(1, 3 * D)

    qkv = pl.pallas_call(
        _qkv_body,
        grid=(T // TB_QKV,),
        in_specs=[pl.BlockSpec((TB_QKV, D), lambda i: (i, 0)),
                  pl.BlockSpec((D, 3 * D), lambda i: (0, 0)),
                  pl.BlockSpec((1, 3 * D), lambda i: (0, 0))],
        out_specs=pl.BlockSpec((TB_QKV, 3 * D), lambda i: (i, 0)),
        out_shape=jax.ShapeDtypeStruct((T, 3 * D), jnp.float32),
    )(xf, Wqkv, bqkv)

    q = qkv[:, :D].reshape(B, S, H, DH).transpose(0, 2, 1, 3)
    k = qkv[:, D:2 * D].reshape(B, S, H, DH).transpose(0, 2, 1, 3)
    v = qkv[:, 2 * D:].reshape(B, S, H, DH).transpose(0, 2, 1, 3)

    o = pl.pallas_call(
        _attn_body,
        grid=(B, H, S // BQ),
        in_specs=[pl.BlockSpec((1, 1, BQ, DH), lambda b, h, i: (b, h, i, 0)),
                  pl.BlockSpec((1, 1, S, DH), lambda b, h, i: (b, h, 0, 0)),
                  pl.BlockSpec((1, 1, S, DH), lambda b, h, i: (b, h, 0, 0))],
        out_specs=pl.BlockSpec((1, 1, BQ, DH), lambda b, h, i: (b, h, i, 0)),
        out_shape=jax.ShapeDtypeStruct((B, H, S, DH), jnp.float32),
    )(q, k, v)

    of = o.transpose(0, 2, 1, 3).reshape(T, D)

    x = pl.pallas_call(
        _oproj_ln_body,
        grid=(NTB,),
        in_specs=[pl.BlockSpec((TB, D), lambda i: (i, 0)),
                  pl.BlockSpec((D, D), lambda i: (0, 0)),
                  pl.BlockSpec((1, D), lambda i: (0, 0)),
                  pl.BlockSpec((TB, D), lambda i: (i, 0)),
                  pl.BlockSpec((1, D), lambda i: (0, 0)),
                  pl.BlockSpec((1, D), lambda i: (0, 0))],
        out_specs=pl.BlockSpec((TB, D), lambda i: (i, 0)),
        out_shape=jax.ShapeDtypeStruct((T, D), jnp.float32),
    )(of, Wo.T, bo.reshape(1, D), xf,
      ln1_g.reshape(1, D), ln1_b.reshape(1, D))

    y_sh = pl.pallas_call(
        _shared_body,
        grid=(NTB,),
        in_specs=[pl.BlockSpec((TB, D), lambda i: (i, 0)),
                  pl.BlockSpec((D, FFN), lambda i: (0, 0)),
                  pl.BlockSpec((1, FFN), lambda i: (0, 0)),
                  pl.BlockSpec((FFN, D), lambda i: (0, 0))],
        out_specs=pl.BlockSpec((TB, D), lambda i: (i, 0)),
        out_shape=jax.ShapeDtypeStruct((T, D), jnp.float32),
    )(x, fc1_W.T, fc1_b.reshape(1, FFN), fc2_W.T)

    gate_spec = pltpu.PrefetchScalarGridSpec(
        num_scalar_prefetch=1,
        grid=(B,),
        in_specs=[pl.BlockSpec((S, D), lambda b, idx: (b, 0)),
                  pl.BlockSpec((1, E, D), lambda b, idx: (idx[b], 0, 0)),
                  pl.BlockSpec((1, 1, E), lambda b, idx: (idx[b], 0, 0))],
        out_specs=[pl.BlockSpec((1, 1, S), lambda b, idx: (b, 0, 0)),
                   pl.BlockSpec((1, 1, S), lambda b, idx: (b, 0, 0))],
    )
    gate, gval = pl.pallas_call(
        _gate_body,
        grid_spec=gate_spec,
        out_shape=[jax.ShapeDtypeStruct((B, 1, S), jnp.int32),
                   jax.ShapeDtypeStruct((B, 1, S), jnp.float32)],
    )(idxes, x, gate_W, gate_b.reshape(ND, 1, E))

    gate_flat = gate.reshape(TROWS, 128)
    dst, bex, src = pl.pallas_call(
        _route_body,
        grid=(NB + 1,),
        in_specs=[pl.BlockSpec((TROWS, 128), lambda j: (0, 0))],
        out_specs=[pl.BlockSpec((TROWS, 128), lambda j: (0, 0)),
                   pl.BlockSpec((1, NB), lambda j: (0, 0)),
                   pl.BlockSpec((1, 1, TBS),
                                lambda j: (jnp.maximum(j - 1, 0), 0, 0))],
        out_shape=[jax.ShapeDtypeStruct((TROWS, 128), jnp.int32),
                   jax.ShapeDtypeStruct((1, NB), jnp.int32),
                   jax.ShapeDtypeStruct((NB, 1, TBS), jnp.int32)],
        scratch_shapes=[pltpu.VMEM((TROWS, 128), jnp.int32)],
    )(gate_flat)

    xs = _sc_gather(x, src.reshape(PAD_T), PAD_T)

    expert_spec = pltpu.PrefetchScalarGridSpec(
        num_scalar_prefetch=1,
        grid=(NB,),
        in_specs=[pl.BlockSpec((TBS, D), lambda i, bx: (i, 0)),
                  pl.BlockSpec((1, INTER, D), lambda i, bx: (bx[i], 0, 0)),
                  pl.BlockSpec((1, 1, INTER), lambda i, bx: (bx[i], 0, 0)),
                  pl.BlockSpec((1, D, INTER), lambda i, bx: (bx[i], 0, 0))],
        out_specs=pl.BlockSpec((TBS, D), lambda i, bx: (i, 0)),
    )
    ys = pl.pallas_call(
        _expert_body,
        grid_spec=expert_spec,
        out_shape=jax.ShapeDtypeStruct((PAD_T, D), jnp.float32),
    )(bex.reshape(NB), xs, exp1_W, exp1_b.reshape(E, 1, INTER), exp2_W)

    ye = _sc_gather(ys, dst.reshape(T), T)

    gval_r = gval.reshape(NTB, 1, TB)
    out = pl.pallas_call(
        _combine_body,
        grid=(NTB,),
        in_specs=[pl.BlockSpec((TB, D), lambda i: (i, 0)),
                  pl.BlockSpec((TB, D), lambda i: (i, 0)),
                  pl.BlockSpec((1, D), lambda i: (0, 0)),
                  pl.BlockSpec((1, 1, TB), lambda i: (i, 0, 0)),
                  pl.BlockSpec((TB, D), lambda i: (i, 0)),
                  pl.BlockSpec((1, D), lambda i: (0, 0)),
                  pl.BlockSpec((1, D), lambda i: (0, 0))],
        out_specs=pl.BlockSpec((TB, D), lambda i: (i, 0)),
        out_shape=jax.ShapeDtypeStruct((T, D), jnp.float32),
    )(y_sh, ye, fc2_b.reshape(1, D), gval_r, x,
      fln_g.reshape(1, D), fln_b.reshape(1, D))

    return out.reshape(B, S, D)


# speedup vs baseline: 1.0447x; 1.0447x over previous
import functools

import jax
import jax.numpy as jnp
from jax.experimental import pallas as pl
from jax.experimental.pallas import tpu as pltpu
from jax.experimental.pallas import tpu_sc as plsc

B, S, D, H = 2, 2048, 768, 12
FFN, INTER, E, ND = 3072, 768, 8, 4
DH = D // H
T = B * S
SCALE = DH ** -0.5

TB_QKV = 512
BQ = 512
TB = 256
NTB = T // TB

TBS = 128
NB = 40
PAD_T = NB * TBS

TROWS = T // 128


def _gelu(x):
    return x * 0.5 * (1.0 + jax.lax.erf(x * (2.0 ** -0.5)))


def _layernorm(t, g, b):
    m = jnp.mean(t, axis=-1, keepdims=True)
    v = jnp.mean((t - m) ** 2, axis=-1, keepdims=True)
    return (t - m) / jnp.sqrt(v + 1e-5) * g + b


def _qkv_body(x_ref, w_ref, b_ref, o_ref):
    o_ref[...] = (jnp.dot(x_ref[...], w_ref[...],
                          preferred_element_type=jnp.float32) + b_ref[...])


def _attn_body(q_ref, k_ref, v_ref, o_ref):
    q = q_ref[0, 0]
    k = k_ref[0, 0]
    v = v_ref[0, 0]
    s = jax.lax.dot_general(q, k, (((1,), (1,)), ((), ())),
                            preferred_element_type=jnp.float32)
    m = jnp.max(s, axis=-1, keepdims=True)
    p = jnp.exp(s - m)
    p = p / jnp.sum(p, axis=-1, keepdims=True)
    o_ref[0, 0] = jnp.dot(p, v, preferred_element_type=jnp.float32)


def _oproj_ln_body(o_ref, w_ref, b_ref, hs_ref, g_ref, bb_ref, x_ref):
    t = (jnp.dot(o_ref[...], w_ref[...], preferred_element_type=jnp.float32)
         + b_ref[...] + hs_ref[...])
    x_ref[...] = _layernorm(t, g_ref[...], bb_ref[...])


def _gate_body(idx_ref, x_ref, gw_ref, gb_ref, gate_ref, gval_ref):
    del idx_ref
    x = x_ref[...]
    gw = gw_ref[0]
    logits = jax.lax.dot_general(x, gw, (((1,), (1,)), ((), ())),
                                 preferred_element_type=jnp.float32)
    logits = logits + gb_ref[0]
    m = jnp.max(logits, axis=-1, keepdims=True)
    p = jnp.exp(logits - m)
    sp = jnp.sum(p, axis=-1)
    gate_ref[0, 0] = jnp.argmax(logits, axis=-1).astype(jnp.int32)
    gval_ref[0, 0] = jnp.max(p, axis=-1) / sp


def _route_body(gate_ref, dst_ref, bex_ref, src_ref, dsts_ref):
    j = pl.program_id(0)

    @pl.when(j == 0)
    def _meta():
        g = gate_ref[...]
        l0 = jax.lax.broadcasted_iota(jnp.int32, (128, 128), 0)
        l1 = jax.lax.broadcasted_iota(jnp.int32, (128, 128), 1)
        slt_lane = (l0 < l1).astype(jnp.float32)
        r0 = jax.lax.broadcasted_iota(jnp.int32, (TROWS, TROWS), 0)
        r1 = jax.lax.broadcasted_iota(jnp.int32, (TROWS, TROWS), 1)
        slt_row = (r1 < r0).astype(jnp.float32)

        dst = jnp.zeros((TROWS, 128), jnp.float32)
        bpos = (jax.lax.broadcasted_iota(jnp.int32, (1, NB), 1)
                * TBS).astype(jnp.float32)
        bex = jnp.zeros((1, NB), jnp.float32)
        base = jnp.float32(0.0)
        for e in range(E):
            oh = (g == e).astype(jnp.float32)
            cs = jnp.dot(oh, slt_lane,
                         preferred_element_type=jnp.float32)
            rs = jnp.sum(oh, axis=1, keepdims=True)
            rp = jnp.dot(slt_row, rs,
                         preferred_element_type=jnp.float32)
            dst = dst + oh * (cs + rp + base)
            cnt = jnp.sum(oh)
            pc = jnp.ceil(cnt * (1.0 / TBS)) * TBS
            bex = bex + jnp.float32(e) * ((bpos >= base) &
                                          (bpos < base + pc)).astype(jnp.float32)
            base = base + pc
        dst_ref[...] = dst.astype(jnp.int32)
        dsts_ref[...] = dst.astype(jnp.int32)
        bex_ref[...] = bex.astype(jnp.int32)
        src_ref[0, 0] = jnp.zeros((TBS,), jnp.int32)

    @pl.when(j > 0)
    def _src():
        p0 = (j - 1) * TBS
        d = dsts_ref[...]
        pos = (jax.lax.broadcasted_iota(jnp.int32, (TBS, TROWS, 128), 0)
               + p0)
        eq = (d[None, :, :] == pos).astype(jnp.float32)
        tok = (jax.lax.broadcasted_iota(jnp.int32, (TBS, TROWS, 128), 1)
               * 128
               + jax.lax.broadcasted_iota(jnp.int32, (TBS, TROWS, 128), 2)
               ).astype(jnp.float32)
        s = jnp.sum(jnp.sum(eq * tok, axis=2), axis=1)
        src_ref[0, 0] = s.astype(jnp.int32)


def _expert_body(bex_ref, xs_ref, e1w_ref, e1b_ref, e2w_ref, ys_ref):
    del bex_ref
    xs = xs_ref[...]
    he = jax.lax.dot_general(xs, e1w_ref[0], (((1,), (1,)), ((), ())),
                             preferred_element_type=jnp.float32)
    he = _gelu(he + e1b_ref[0])
    ys_ref[...] = jax.lax.dot_general(he, e2w_ref[0], (((1,), (1,)), ((), ())),
                                      preferred_element_type=jnp.float32)


def _shared_body(x_ref, w1_ref, b1_ref, w2_ref, o_ref):
    h = jnp.dot(x_ref[...], w1_ref[...], preferred_element_type=jnp.float32)
    h = _gelu(h + b1_ref[...])
    o_ref[...] = jnp.dot(h, w2_ref[...], preferred_element_type=jnp.float32)


def _combine_body(ysh_ref, ye_ref, b2_ref, gval_ref, x_ref, g_ref, bb_ref,
                  o_ref):
    x = x_ref[...]
    y = ysh_ref[...] + ye_ref[...] + b2_ref[...]
    t = y * gval_ref[0, 0][:, None] + x
    o_ref[...] = _layernorm(t, g_ref[...], bb_ref[...])


def _sc_gather(table, idx, n_rows):
    info = plsc.get_sparse_core_info()
    nc, ns = info.num_cores, info.num_subcores
    nw = nc * ns
    per_w = n_rows // nw
    ch = per_w // 2
    n_ch = 2
    mesh = plsc.VectorSubcoreMesh(core_axis_name="c", subcore_axis_name="s")

    @functools.partial(
        pl.kernel, mesh=mesh,
        out_type=jax.ShapeDtypeStruct((n_rows, D), jnp.float32),
        scratch_types=[pltpu.VMEM((per_w,), jnp.int32),
                       pltpu.VMEM((ch, D), jnp.float32),
                       pltpu.VMEM((ch, D), jnp.float32),
                       pltpu.SemaphoreType.DMA,
                       pltpu.SemaphoreType.DMA])
    def gk(table_hbm, idx_hbm, out_hbm, idx_v, b0, b1, gsem, ssem):
        bufs = [b0, b1]
        wid = jax.lax.axis_index("s") * nc + jax.lax.axis_index("c")
        base = wid * per_w
        pltpu.sync_copy(idx_hbm.at[pl.ds(base, per_w)], idx_v)
        gathers = [pltpu.async_copy(table_hbm.at[idx_v.at[pl.ds(0, ch)]],
                                    b0, gsem)]
        stores = []
        for c in range(n_ch):
            if c + 1 < n_ch:
                if c >= 1:
                    stores[c - 1].wait()
                gathers.append(pltpu.async_copy(
                    table_hbm.at[idx_v.at[pl.ds((c + 1) * ch, ch)]],
                    bufs[(c + 1) % 2], gsem))
            gathers[c].wait()
            stores.append(pltpu.async_copy(
                bufs[c % 2], out_hbm.at[pl.ds(base + c * ch, ch)], ssem))
        stores[-1].wait()
        if n_ch >= 2:
            stores[-2].wait()

    return gk(table, idx)


def kernel(hidden_states, idxes, Wq, bq, Wk, bk, Wv, bv, Wo, bo, ln1_g, ln1_b,
           fc1_W, fc1_b, fc2_W, fc2_b, exp1_W, exp1_b, exp2_W, gate_W, gate_b,
           fln_g, fln_b):
    xf = hidden_states.reshape(T, D)
    Wqkv = jnp.concatenate([Wq.T * SCALE, Wk.T, Wv.T], axis=1)
    bqkv = jnp.concatenate([bq * SCALE, bk, bv]).reshape(1, 3 * D)

    qkv = pl.pallas_call(
        _qkv_body,
        grid=(T // TB_QKV,),
        in_specs=[pl.BlockSpec((TB_QKV, D), lambda i: (i, 0)),
                  pl.BlockSpec((D, 3 * D), lambda i: (0, 0)),
                  pl.BlockSpec((1, 3 * D), lambda i: (0, 0))],
        out_specs=pl.BlockSpec((TB_QKV, 3 * D), lambda i: (i, 0)),
        out_shape=jax.ShapeDtypeStruct((T, 3 * D), jnp.float32),
    )(xf, Wqkv, bqkv)

    q = qkv[:, :D].reshape(B, S, H, DH).transpose(0, 2, 1, 3)
    k = qkv[:, D:2 * D].reshape(B, S, H, DH).transpose(0, 2, 1, 3)
    v = qkv[:, 2 * D:].reshape(B, S, H, DH).transpose(0, 2, 1, 3)

    o = pl.pallas_call(
        _attn_body,
        grid=(B, H, S // BQ),
        in_specs=[pl.BlockSpec((1, 1, BQ, DH), lambda b, h, i: (b, h, i, 0)),
                  pl.BlockSpec((1, 1, S, DH), lambda b, h, i: (b, h, 0, 0)),
                  pl.BlockSpec((1, 1, S, DH), lambda b, h, i: (b, h, 0, 0))],
        out_specs=pl.BlockSpec((1, 1, BQ, DH), lambda b, h, i: (b, h, i, 0)),
        out_shape=jax.ShapeDtypeStruct((B, H, S, DH), jnp.float32),
    )(q, k, v)

    of = o.transpose(0, 2, 1, 3).reshape(T, D)

    x = pl.pallas_call(
        _oproj_ln_body,
        grid=(NTB,),
        in_specs=[pl.BlockSpec((TB, D), lambda i: (i, 0)),
                  pl.BlockSpec((D, D), lambda i: (0, 0)),
                  pl.BlockSpec((1, D), lambda i: (0, 0)),
                  pl.BlockSpec((TB, D), lambda i: (i, 0)),
                  pl.BlockSpec((1, D), lambda i: (0, 0)),
                  pl.BlockSpec((1, D), lambda i: (0, 0))],
        out_specs=pl.BlockSpec((TB, D), lambda i: (i, 0)),
        out_shape=jax.ShapeDtypeStruct((T, D), jnp.float32),
    )(of, Wo.T, bo.reshape(1, D), xf,
      ln1_g.reshape(1, D), ln1_b.reshape(1, D))

    y_sh = pl.pallas_call(
        _shared_body,
        grid=(NTB,),
        in_specs=[pl.BlockSpec((TB, D), lambda i: (i, 0)),
                  pl.BlockSpec((D, FFN), lambda i: (0, 0)),
                  pl.BlockSpec((1, FFN), lambda i: (0, 0)),
                  pl.BlockSpec((FFN, D), lambda i: (0, 0))],
        out_specs=pl.BlockSpec((TB, D), lambda i: (i, 0)),
        out_shape=jax.ShapeDtypeStruct((T, D), jnp.float32),
    )(x, fc1_W.T, fc1_b.reshape(1, FFN), fc2_W.T)

    gate_spec = pltpu.PrefetchScalarGridSpec(
        num_scalar_prefetch=1,
        grid=(B,),
        in_specs=[pl.BlockSpec((S, D), lambda b, idx: (b, 0)),
                  pl.BlockSpec((1, E, D), lambda b, idx: (idx[b], 0, 0)),
                  pl.BlockSpec((1, 1, E), lambda b, idx: (idx[b], 0, 0))],
        out_specs=[pl.BlockSpec((1, 1, S), lambda b, idx: (b, 0, 0)),
                   pl.BlockSpec((1, 1, S), lambda b, idx: (b, 0, 0))],
    )
    gate, gval = pl.pallas_call(
        _gate_body,
        grid_spec=gate_spec,
        out_shape=[jax.ShapeDtypeStruct((B, 1, S), jnp.int32),
                   jax.ShapeDtypeStruct((B, 1, S), jnp.float32)],
    )(idxes, x, gate_W, gate_b.reshape(ND, 1, E))

    gate_flat = gate.reshape(TROWS, 128)
    dst, bex, src = pl.pallas_call(
        _route_body,
        grid=(NB + 1,),
        in_specs=[pl.BlockSpec((TROWS, 128), lambda j: (0, 0))],
        out_specs=[pl.BlockSpec((TROWS, 128), lambda j: (0, 0)),
                   pl.BlockSpec((1, NB), lambda j: (0, 0)),
                   pl.BlockSpec((1, 1, TBS),
                                lambda j: (jnp.maximum(j - 1, 0), 0, 0))],
        out_shape=[jax.ShapeDtypeStruct((TROWS, 128), jnp.int32),
                   jax.ShapeDtypeStruct((1, NB), jnp.int32),
                   jax.ShapeDtypeStruct((NB, 1, TBS), jnp.int32)],
        scratch_shapes=[pltpu.VMEM((TROWS, 128), jnp.int32)],
    )(gate_flat)

    xs = _sc_gather(x, src.reshape(PAD_T), PAD_T)

    expert_spec = pltpu.PrefetchScalarGridSpec(
        num_scalar_prefetch=1,
        grid=(NB,),
        in_specs=[pl.BlockSpec((TBS, D), lambda i, bx: (i, 0)),
                  pl.BlockSpec((1, INTER, D), lambda i, bx: (bx[i], 0, 0)),
                  pl.BlockSpec((1, 1, INTER), lambda i, bx: (bx[i], 0, 0)),
                  pl.BlockSpec((1, D, INTER), lambda i, bx: (bx[i], 0, 0))],
        out_specs=pl.BlockSpec((TBS, D), lambda i, bx: (i, 0)),
    )
    ys = pl.pallas_call(
        _expert_body,
        grid_spec=expert_spec,
        out_shape=jax.ShapeDtypeStruct((PAD_T, D), jnp.float32),
    )(bex.reshape(NB), xs, exp1_W, exp1_b.reshape(E, 1, INTER), exp2_W)

    ye = _sc_gather(ys, dst.reshape(T), T)

    gval_r = gval.reshape(NTB, 1, TB)
    out = pl.pallas_call(
        _combine_body,
        grid=(NTB,),
        in_specs=[pl.BlockSpec((TB, D), lambda i: (i, 0)),
                  pl.BlockSpec((TB, D), lambda i: (i, 0)),
                  pl.BlockSpec((1, D), lambda i: (0, 0)),
                  pl.BlockSpec((1, 1, TB), lambda i: (i, 0, 0)),
                  pl.BlockSpec((TB, D), lambda i: (i, 0)),
                  pl.BlockSpec((1, D), lambda i: (0, 0)),
                  pl.BlockSpec((1, D), lambda i: (0, 0))],
        out_specs=pl.BlockSpec((TB, D), lambda i: (i, 0)),
        out_shape=jax.ShapeDtypeStruct((T, D), jnp.float32),
    )(y_sh, ye, fc2_b.reshape(1, D), gval_r, x,
      fln_g.reshape(1, D), fln_b.reshape(1, D))

    return out.reshape(B, S, D)


# speedup vs baseline: 1.0825x; 1.0361x over previous
import functools

import jax
import jax.numpy as jnp
from jax.experimental import pallas as pl
from jax.experimental.pallas import tpu as pltpu
from jax.experimental.pallas import tpu_sc as plsc

B, S, D, H = 2, 2048, 768, 12
FFN, INTER, E, ND = 3072, 768, 8, 4
DH = D // H
T = B * S
SCALE = DH ** -0.5

TB_QKV = 512
BQ = 512
TB = 256
NTB = T // TB

TBS = 128
NB = 40
PAD_T = NB * TBS

TROWS = T // 128


def _gelu(x):
    return x * 0.5 * (1.0 + jax.lax.erf(x * (2.0 ** -0.5)))


def _layernorm(t, g, b):
    m = jnp.mean(t, axis=-1, keepdims=True)
    v = jnp.mean((t - m) ** 2, axis=-1, keepdims=True)
    return (t - m) / jnp.sqrt(v + 1e-5) * g + b


def _qkv_body(x_ref, w_ref, b_ref, o_ref):
    o_ref[...] = (jnp.dot(x_ref[...], w_ref[...],
                          preferred_element_type=jnp.float32) + b_ref[...])


def _attn_body(q_ref, k_ref, v_ref, o_ref):
    q = q_ref[0, 0]
    k = k_ref[0, 0]
    v = v_ref[0, 0]
    s = jax.lax.dot_general(q, k, (((1,), (1,)), ((), ())),
                            preferred_element_type=jnp.float32)
    m = jnp.max(s, axis=-1, keepdims=True)
    p = jnp.exp(s - m)
    p = p / jnp.sum(p, axis=-1, keepdims=True)
    o_ref[0, 0] = jnp.dot(p, v, preferred_element_type=jnp.float32)


def _oproj_ln_body(o_ref, w_ref, b_ref, hs_ref, g_ref, bb_ref, x_ref):
    t = (jnp.dot(o_ref[...], w_ref[...], preferred_element_type=jnp.float32)
         + b_ref[...] + hs_ref[...])
    x_ref[...] = _layernorm(t, g_ref[...], bb_ref[...])


def _gate_body(idx_ref, x_ref, gw_ref, gb_ref, gate_ref, gval_ref):
    del idx_ref
    x = x_ref[...]
    gw = gw_ref[0]
    logits = jax.lax.dot_general(x, gw, (((1,), (1,)), ((), ())),
                                 preferred_element_type=jnp.float32)
    logits = logits + gb_ref[0]
    m = jnp.max(logits, axis=-1, keepdims=True)
    p = jnp.exp(logits - m)
    sp = jnp.sum(p, axis=-1)
    gate_ref[0, 0] = jnp.argmax(logits, axis=-1).astype(jnp.int32)
    gval_ref[0, 0] = jnp.max(p, axis=-1) / sp


def _route_body(gate_ref, dst_ref, bex_ref, src_ref, dsts_ref):
    j = pl.program_id(0)

    @pl.when(j == 0)
    def _meta():
        g = gate_ref[...]
        l0 = jax.lax.broadcasted_iota(jnp.int32, (128, 128), 0)
        l1 = jax.lax.broadcasted_iota(jnp.int32, (128, 128), 1)
        slt_lane = (l0 < l1).astype(jnp.float32)
        r0 = jax.lax.broadcasted_iota(jnp.int32, (TROWS, TROWS), 0)
        r1 = jax.lax.broadcasted_iota(jnp.int32, (TROWS, TROWS), 1)
        slt_row = (r1 < r0).astype(jnp.float32)

        dst = jnp.zeros((TROWS, 128), jnp.float32)
        bpos = (jax.lax.broadcasted_iota(jnp.int32, (1, NB), 1)
                * TBS).astype(jnp.float32)
        bex = jnp.zeros((1, NB), jnp.float32)
        base = jnp.float32(0.0)
        for e in range(E):
            oh = (g == e).astype(jnp.float32)
            cs = jnp.dot(oh, slt_lane,
                         preferred_element_type=jnp.float32)
            rs = jnp.sum(oh, axis=1, keepdims=True)
            rp = jnp.dot(slt_row, rs,
                         preferred_element_type=jnp.float32)
            dst = dst + oh * (cs + rp + base)
            cnt = jnp.sum(oh)
            pc = jnp.ceil(cnt * (1.0 / TBS)) * TBS
            bex = bex + jnp.float32(e) * ((bpos >= base) &
                                          (bpos < base + pc)).astype(jnp.float32)
            base = base + pc
        dst_ref[...] = dst.astype(jnp.int32)
        dsts_ref[...] = dst.astype(jnp.int32)
        bex_ref[...] = bex.astype(jnp.int32)
        src_ref[0, 0] = jnp.zeros((TBS,), jnp.int32)

    @pl.when(j > 0)
    def _src():
        p0 = (j - 1) * TBS
        d = dsts_ref[...]
        pos = (jax.lax.broadcasted_iota(jnp.int32, (TBS, TROWS, 128), 0)
               + p0)
        eq = (d[None, :, :] == pos).astype(jnp.float32)
        tok = (jax.lax.broadcasted_iota(jnp.int32, (TBS, TROWS, 128), 1)
               * 128
               + jax.lax.broadcasted_iota(jnp.int32, (TBS, TROWS, 128), 2)
               ).astype(jnp.float32)
        s = jnp.sum(jnp.sum(eq * tok, axis=2), axis=1)
        hit = jnp.sum(jnp.sum(eq, axis=2), axis=1)
        pv = jax.lax.broadcasted_iota(jnp.int32, (TBS,), 0) + p0
        fb = (pv - T * (pv >= T)).astype(jnp.float32)
        src_ref[0, 0] = (s + (1.0 - hit) * fb).astype(jnp.int32)


def _expert_body(bex_ref, xs_ref, e1w_ref, e1b_ref, e2w_ref, ys_ref):
    del bex_ref
    xs = xs_ref[...]
    he = jax.lax.dot_general(xs, e1w_ref[0], (((1,), (1,)), ((), ())),
                             preferred_element_type=jnp.float32)
    he = _gelu(he + e1b_ref[0])
    ys_ref[...] = jax.lax.dot_general(he, e2w_ref[0], (((1,), (1,)), ((), ())),
                                      preferred_element_type=jnp.float32)


def _shared_body(x_ref, w1_ref, b1_ref, w2_ref, o_ref):
    h = jnp.dot(x_ref[...], w1_ref[...], preferred_element_type=jnp.float32)
    h = _gelu(h + b1_ref[...])
    o_ref[...] = jnp.dot(h, w2_ref[...], preferred_element_type=jnp.float32)


def _combine_body(ysh_ref, ye_ref, b2_ref, gval_ref, x_ref, g_ref, bb_ref,
                  o_ref):
    x = x_ref[...]
    y = ysh_ref[...] + ye_ref[...] + b2_ref[...]
    t = y * gval_ref[0, 0][:, None] + x
    o_ref[...] = _layernorm(t, g_ref[...], bb_ref[...])


def _sc_gather(table, idx, n_rows):
    info = plsc.get_sparse_core_info()
    nc, ns = info.num_cores, info.num_subcores
    nw = nc * ns
    per_w = n_rows // nw
    n_ch = 4
    ch = per_w // n_ch
    mesh = plsc.VectorSubcoreMesh(core_axis_name="c", subcore_axis_name="s")

    @functools.partial(
        pl.kernel, mesh=mesh,
        out_type=jax.ShapeDtypeStruct((n_rows, D), jnp.float32),
        scratch_types=[pltpu.VMEM((per_w,), jnp.int32)]
        + [pltpu.VMEM((ch, D), jnp.float32) for _ in range(4)]
        + [pltpu.SemaphoreType.DMA, pltpu.SemaphoreType.DMA])
    def gk(table_hbm, idx_hbm, out_hbm, idx_v, b0, b1, b2, b3, gsem, ssem):
        bufs = [b0, b1, b2, b3]
        wid = jax.lax.axis_index("s") * nc + jax.lax.axis_index("c")
        base = wid * per_w
        pltpu.sync_copy(idx_hbm.at[pl.ds(base, per_w)], idx_v)
        gathers = [pltpu.async_copy(
            table_hbm.at[idx_v.at[pl.ds(c * ch, ch)]], bufs[c], gsem)
            for c in range(n_ch)]
        stores = []
        for c in range(n_ch):
            gathers[c].wait()
            stores.append(pltpu.async_copy(
                bufs[c], out_hbm.at[pl.ds(base + c * ch, ch)], ssem))
        for cp in stores:
            cp.wait()

    return gk(table, idx)


def kernel(hidden_states, idxes, Wq, bq, Wk, bk, Wv, bv, Wo, bo, ln1_g, ln1_b,
           fc1_W, fc1_b, fc2_W, fc2_b, exp1_W, exp1_b, exp2_W, gate_W, gate_b,
           fln_g, fln_b):
    xf = hidden_states.reshape(T, D)
    Wqkv = jnp.concatenate([Wq.T * SCALE, Wk.T, Wv.T], axis=1)
    bqkv = jnp.concatenate([bq * SCALE, bk, bv]).reshape(1, 3 * D)

    qkv = pl.pallas_call(
        _qkv_body,
        grid=(T // TB_QKV,),
        in_specs=[pl.BlockSpec((TB_QKV, D), lambda i: (i, 0)),
                  pl.BlockSpec((D, 3 * D), lambda i: (0, 0)),
                  pl.BlockSpec((1, 3 * D), lambda i: (0, 0))],
        out_specs=pl.BlockSpec((TB_QKV, 3 * D), lambda i: (i, 0)),
        out_shape=jax.ShapeDtypeStruct((T, 3 * D), jnp.float32),
    )(xf, Wqkv, bqkv)

    q = qkv[:, :D].reshape(B, S, H, DH).transpose(0, 2, 1, 3)
    k = qkv[:, D:2 * D].reshape(B, S, H, DH).transpose(0, 2, 1, 3)
    v = qkv[:, 2 * D:].reshape(B, S, H, DH).transpose(0, 2, 1, 3)

    o = pl.pallas_call(
        _attn_body,
        grid=(B, H, S // BQ),
        in_specs=[pl.BlockSpec((1, 1, BQ, DH), lambda b, h, i: (b, h, i, 0)),
                  pl.BlockSpec((1, 1, S, DH), lambda b, h, i: (b, h, 0, 0)),
                  pl.BlockSpec((1, 1, S, DH), lambda b, h, i: (b, h, 0, 0))],
        out_specs=pl.BlockSpec((1, 1, BQ, DH), lambda b, h, i: (b, h, i, 0)),
        out_shape=jax.ShapeDtypeStruct((B, H, S, DH), jnp.float32),
    )(q, k, v)

    of = o.transpose(0, 2, 1, 3).reshape(T, D)

    x = pl.pallas_call(
        _oproj_ln_body,
        grid=(NTB,),
        in_specs=[pl.BlockSpec((TB, D), lambda i: (i, 0)),
                  pl.BlockSpec((D, D), lambda i: (0, 0)),
                  pl.BlockSpec((1, D), lambda i: (0, 0)),
                  pl.BlockSpec((TB, D), lambda i: (i, 0)),
                  pl.BlockSpec((1, D), lambda i: (0, 0)),
                  pl.BlockSpec((1, D), lambda i: (0, 0))],
        out_specs=pl.BlockSpec((TB, D), lambda i: (i, 0)),
        out_shape=jax.ShapeDtypeStruct((T, D), jnp.float32),
    )(of, Wo.T, bo.reshape(1, D), xf,
      ln1_g.reshape(1, D), ln1_b.reshape(1, D))

    y_sh = pl.pallas_call(
        _shared_body,
        grid=(NTB,),
        in_specs=[pl.BlockSpec((TB, D), lambda i: (i, 0)),
                  pl.BlockSpec((D, FFN), lambda i: (0, 0)),
                  pl.BlockSpec((1, FFN), lambda i: (0, 0)),
                  pl.BlockSpec((FFN, D), lambda i: (0, 0))],
        out_specs=pl.BlockSpec((TB, D), lambda i: (i, 0)),
        out_shape=jax.ShapeDtypeStruct((T, D), jnp.float32),
    )(x, fc1_W.T, fc1_b.reshape(1, FFN), fc2_W.T)

    gate_spec = pltpu.PrefetchScalarGridSpec(
        num_scalar_prefetch=1,
        grid=(B,),
        in_specs=[pl.BlockSpec((S, D), lambda b, idx: (b, 0)),
                  pl.BlockSpec((1, E, D), lambda b, idx: (idx[b], 0, 0)),
                  pl.BlockSpec((1, 1, E), lambda b, idx: (idx[b], 0, 0))],
        out_specs=[pl.BlockSpec((1, 1, S), lambda b, idx: (b, 0, 0)),
                   pl.BlockSpec((1, 1, S), lambda b, idx: (b, 0, 0))],
    )
    gate, gval = pl.pallas_call(
        _gate_body,
        grid_spec=gate_spec,
        out_shape=[jax.ShapeDtypeStruct((B, 1, S), jnp.int32),
                   jax.ShapeDtypeStruct((B, 1, S), jnp.float32)],
    )(idxes, x, gate_W, gate_b.reshape(ND, 1, E))

    gate_flat = gate.reshape(TROWS, 128)
    dst, bex, src = pl.pallas_call(
        _route_body,
        grid=(NB + 1,),
        in_specs=[pl.BlockSpec((TROWS, 128), lambda j: (0, 0))],
        out_specs=[pl.BlockSpec((TROWS, 128), lambda j: (0, 0)),
                   pl.BlockSpec((1, NB), lambda j: (0, 0)),
                   pl.BlockSpec((1, 1, TBS),
                                lambda j: (jnp.maximum(j - 1, 0), 0, 0))],
        out_shape=[jax.ShapeDtypeStruct((TROWS, 128), jnp.int32),
                   jax.ShapeDtypeStruct((1, NB), jnp.int32),
                   jax.ShapeDtypeStruct((NB, 1, TBS), jnp.int32)],
        scratch_shapes=[pltpu.VMEM((TROWS, 128), jnp.int32)],
    )(gate_flat)

    xs = _sc_gather(x, src.reshape(PAD_T), PAD_T)

    expert_spec = pltpu.PrefetchScalarGridSpec(
        num_scalar_prefetch=1,
        grid=(NB,),
        in_specs=[pl.BlockSpec((TBS, D), lambda i, bx: (i, 0)),
                  pl.BlockSpec((1, INTER, D), lambda i, bx: (bx[i], 0, 0)),
                  pl.BlockSpec((1, 1, INTER), lambda i, bx: (bx[i], 0, 0)),
                  pl.BlockSpec((1, D, INTER), lambda i, bx: (bx[i], 0, 0))],
        out_specs=pl.BlockSpec((TBS, D), lambda i, bx: (i, 0)),
    )
    ys = pl.pallas_call(
        _expert_body,
        grid_spec=expert_spec,
        out_shape=jax.ShapeDtypeStruct((PAD_T, D), jnp.float32),
    )(bex.reshape(NB), xs, exp1_W, exp1_b.reshape(E, 1, INTER), exp2_W)

    ye = _sc_gather(ys, dst.reshape(T), T)

    gval_r = gval.reshape(NTB, 1, TB)
    out = pl.pallas_call(
        _combine_body,
        grid=(NTB,),
        in_specs=[pl.BlockSpec((TB, D), lambda i: (i, 0)),
                  pl.BlockSpec((TB, D), lambda i: (i, 0)),
                  pl.BlockSpec((1, D), lambda i: (0, 0)),
                  pl.BlockSpec((1, 1, TB), lambda i: (i, 0, 0)),
                  pl.BlockSpec((TB, D), lambda i: (i, 0)),
                  pl.BlockSpec((1, D), lambda i: (0, 0)),
                  pl.BlockSpec((1, D), lambda i: (0, 0))],
        out_specs=pl.BlockSpec((TB, D), lambda i: (i, 0)),
        out_shape=jax.ShapeDtypeStruct((T, D), jnp.float32),
    )(y_sh, ye, fc2_b.reshape(1, D), gval_r, x,
      fln_g.reshape(1, D), fln_b.reshape(1, D))

    return out.reshape(B, S, D)


# speedup vs baseline: 1.4152x; 1.3073x over previous
import functools

import jax
import jax.numpy as jnp
from jax.experimental import pallas as pl
from jax.experimental.pallas import tpu as pltpu
from jax.experimental.pallas import tpu_sc as plsc

B, S, D, H = 2, 2048, 768, 12
FFN, INTER, E, ND = 3072, 768, 8, 4
DH = D // H
T = B * S
SCALE = DH ** -0.5

TB_QKV = 512
BQ = 512
TB = 256
NTB = T // TB

TBS = 128
NB = 40
PAD_T = NB * TBS

TROWS = T // 128


def _gelu(x):
    return x * 0.5 * (1.0 + jax.lax.erf(x * (2.0 ** -0.5)))


def _layernorm(t, g, b):
    m = jnp.mean(t, axis=-1, keepdims=True)
    v = jnp.mean((t - m) ** 2, axis=-1, keepdims=True)
    return (t - m) / jnp.sqrt(v + 1e-5) * g + b


def _qkv_body(x_ref, wq_ref, wk_ref, wv_ref, b_ref, q_ref, k_ref, v_ref):
    x = x_ref[...]
    cn = (((1,), (1,)), ((), ()))
    q = (jax.lax.dot_general(x, wq_ref[...], cn,
                             preferred_element_type=jnp.float32)
         + b_ref[:, :D]) * SCALE
    k = (jax.lax.dot_general(x, wk_ref[...], cn,
                             preferred_element_type=jnp.float32)
         + b_ref[:, D:2 * D])
    v = (jax.lax.dot_general(x, wv_ref[...], cn,
                             preferred_element_type=jnp.float32)
         + b_ref[:, 2 * D:])
    q_ref[0] = jnp.stack([q[:, h * DH:(h + 1) * DH] for h in range(H)], 0)
    k_ref[0] = jnp.stack([k[:, h * DH:(h + 1) * DH] for h in range(H)], 0)
    v_ref[0] = jnp.stack([v[:, h * DH:(h + 1) * DH] for h in range(H)], 0)


def _attn_body(q_ref, k_ref, v_ref, o_ref):
    q = q_ref[0, 0]
    k = k_ref[0, 0]
    v = v_ref[0, 0]
    s = jax.lax.dot_general(q, k, (((1,), (1,)), ((), ())),
                            preferred_element_type=jnp.float32)
    m = jnp.max(s, axis=-1, keepdims=True)
    p = jnp.exp(s - m)
    p = p / jnp.sum(p, axis=-1, keepdims=True)
    o_ref[0, 0] = jnp.dot(p, v, preferred_element_type=jnp.float32)


def _oproj_ln_body(o_ref, w_ref, b_ref, hs_ref, g_ref, bb_ref, x_ref):
    o = jnp.concatenate([o_ref[0, h] for h in range(H)], axis=-1)
    t = (jax.lax.dot_general(o, w_ref[...], (((1,), (1,)), ((), ())),
                             preferred_element_type=jnp.float32)
         + b_ref[...] + hs_ref[...])
    x_ref[...] = _layernorm(t, g_ref[...], bb_ref[...])


def _gate_body(idx_ref, x_ref, gw_ref, gb_ref, gate_ref, gval_ref):
    del idx_ref
    x = x_ref[...]
    gw = gw_ref[0]
    logits = jax.lax.dot_general(x, gw, (((1,), (1,)), ((), ())),
                                 preferred_element_type=jnp.float32)
    logits = logits + gb_ref[0]
    m = jnp.max(logits, axis=-1, keepdims=True)
    p = jnp.exp(logits - m)
    sp = jnp.sum(p, axis=-1)
    gate_ref[0, 0] = jnp.argmax(logits, axis=-1).astype(jnp.int32)
    gval_ref[0, 0] = jnp.max(p, axis=-1) / sp


def _route_body(gate_ref, dst_ref, bex_ref, src_ref, dsts_ref):
    j = pl.program_id(0)

    @pl.when(j == 0)
    def _meta():
        g = gate_ref[...]
        l0 = jax.lax.broadcasted_iota(jnp.int32, (128, 128), 0)
        l1 = jax.lax.broadcasted_iota(jnp.int32, (128, 128), 1)
        slt_lane = (l0 < l1).astype(jnp.float32)
        r0 = jax.lax.broadcasted_iota(jnp.int32, (TROWS, TROWS), 0)
        r1 = jax.lax.broadcasted_iota(jnp.int32, (TROWS, TROWS), 1)
        slt_row = (r1 < r0).astype(jnp.float32)

        dst = jnp.zeros((TROWS, 128), jnp.float32)
        bpos = (jax.lax.broadcasted_iota(jnp.int32, (1, NB), 1)
                * TBS).astype(jnp.float32)
        bex = jnp.zeros((1, NB), jnp.float32)
        base = jnp.float32(0.0)
        for e in range(E):
            oh = (g == e).astype(jnp.float32)
            cs = jnp.dot(oh, slt_lane,
                         preferred_element_type=jnp.float32)
            rs = jnp.sum(oh, axis=1, keepdims=True)
            rp = jnp.dot(slt_row, rs,
                         preferred_element_type=jnp.float32)
            dst = dst + oh * (cs + rp + base)
            cnt = jnp.sum(oh)
            pc = jnp.ceil(cnt * (1.0 / TBS)) * TBS
            bex = bex + jnp.float32(e) * ((bpos >= base) &
                                          (bpos < base + pc)).astype(jnp.float32)
            base = base + pc
        dst_ref[...] = dst.astype(jnp.int32)
        dsts_ref[...] = dst.astype(jnp.int32)
        bex_ref[...] = bex.astype(jnp.int32)
        src_ref[0, 0] = jnp.zeros((TBS,), jnp.int32)

    @pl.when(j > 0)
    def _src():
        p0 = (j - 1) * TBS
        d = dsts_ref[...]
        pos = (jax.lax.broadcasted_iota(jnp.int32, (TBS, TROWS, 128), 0)
               + p0)
        eq = (d[None, :, :] == pos).astype(jnp.float32)
        tok = (jax.lax.broadcasted_iota(jnp.int32, (TBS, TROWS, 128), 1)
               * 128
               + jax.lax.broadcasted_iota(jnp.int32, (TBS, TROWS, 128), 2)
               ).astype(jnp.float32)
        s = jnp.sum(jnp.sum(eq * tok, axis=2), axis=1)
        hit = jnp.sum(jnp.sum(eq, axis=2), axis=1)
        pv = jax.lax.broadcasted_iota(jnp.int32, (TBS,), 0) + p0
        fb = (pv - T * (pv >= T)).astype(jnp.float32)
        src_ref[0, 0] = (s + (1.0 - hit) * fb).astype(jnp.int32)


def _expert_body(bex_ref, xs_ref, e1w_ref, e1b_ref, e2w_ref, ys_ref):
    del bex_ref
    xs = xs_ref[...]
    he = jax.lax.dot_general(xs, e1w_ref[0], (((1,), (1,)), ((), ())),
                             preferred_element_type=jnp.float32)
    he = _gelu(he + e1b_ref[0])
    ys_ref[...] = jax.lax.dot_general(he, e2w_ref[0], (((1,), (1,)), ((), ())),
                                      preferred_element_type=jnp.float32)


def _shared_body(x_ref, w1_ref, b1_ref, w2_ref, o_ref):
    cn = (((1,), (1,)), ((), ()))
    h = jax.lax.dot_general(x_ref[...], w1_ref[...], cn,
                            preferred_element_type=jnp.float32)
    h = _gelu(h + b1_ref[...])
    o_ref[...] = jax.lax.dot_general(h, w2_ref[...], cn,
                                     preferred_element_type=jnp.float32)


def _combine_body(ysh_ref, ye_ref, b2_ref, gval_ref, x_ref, g_ref, bb_ref,
                  o_ref):
    x = x_ref[...]
    y = ysh_ref[...] + ye_ref[...] + b2_ref[...]
    t = y * gval_ref[0, 0][:, None] + x
    o_ref[...] = _layernorm(t, g_ref[...], bb_ref[...])


def _sc_gather(table, idx, n_rows):
    info = plsc.get_sparse_core_info()
    nc, ns = info.num_cores, info.num_subcores
    nw = nc * ns
    per_w = n_rows // nw
    n_ch = 4
    ch = per_w // n_ch
    mesh = plsc.VectorSubcoreMesh(core_axis_name="c", subcore_axis_name="s")

    @functools.partial(
        pl.kernel, mesh=mesh,
        out_type=jax.ShapeDtypeStruct((n_rows, D), jnp.float32),
        scratch_types=[pltpu.VMEM((per_w,), jnp.int32)]
        + [pltpu.VMEM((ch, D), jnp.float32) for _ in range(4)]
        + [pltpu.SemaphoreType.DMA, pltpu.SemaphoreType.DMA])
    def gk(table_hbm, idx_hbm, out_hbm, idx_v, b0, b1, b2, b3, gsem, ssem):
        bufs = [b0, b1, b2, b3]
        wid = jax.lax.axis_index("s") * nc + jax.lax.axis_index("c")
        base = wid * per_w
        pltpu.sync_copy(idx_hbm.at[pl.ds(base, per_w)], idx_v)
        gathers = [pltpu.async_copy(
            table_hbm.at[idx_v.at[pl.ds(c * ch, ch)]], bufs[c], gsem)
            for c in range(n_ch)]
        stores = []
        for c in range(n_ch):
            gathers[c].wait()
            stores.append(pltpu.async_copy(
                bufs[c], out_hbm.at[pl.ds(base + c * ch, ch)], ssem))
        for cp in stores:
            cp.wait()

    return gk(table, idx)


def kernel(hidden_states, idxes, Wq, bq, Wk, bk, Wv, bv, Wo, bo, ln1_g, ln1_b,
           fc1_W, fc1_b, fc2_W, fc2_b, exp1_W, exp1_b, exp2_W, gate_W, gate_b,
           fln_g, fln_b):
    xf = hidden_states.reshape(T, D)
    bqkv = jnp.concatenate([bq, bk, bv]).reshape(1, 3 * D)
    sb = S // TB_QKV

    hspec = pl.BlockSpec((1, H, TB_QKV, DH), lambda i: (i // sb, 0, i % sb, 0))
    q, k, v = pl.pallas_call(
        _qkv_body,
        grid=(T // TB_QKV,),
        in_specs=[pl.BlockSpec((TB_QKV, D), lambda i: (i, 0)),
                  pl.BlockSpec((D, D), lambda i: (0, 0)),
                  pl.BlockSpec((D, D), lambda i: (0, 0)),
                  pl.BlockSpec((D, D), lambda i: (0, 0)),
                  pl.BlockSpec((1, 3 * D), lambda i: (0, 0))],
        out_specs=[hspec, hspec, hspec],
        out_shape=[jax.ShapeDtypeStruct((B, H, S, DH), jnp.float32)] * 3,
    )(xf, Wq, Wk, Wv, bqkv)

    o = pl.pallas_call(
        _attn_body,
        grid=(B, H, S // BQ),
        in_specs=[pl.BlockSpec((1, 1, BQ, DH), lambda b, h, i: (b, h, i, 0)),
                  pl.BlockSpec((1, 1, S, DH), lambda b, h, i: (b, h, 0, 0)),
                  pl.BlockSpec((1, 1, S, DH), lambda b, h, i: (b, h, 0, 0))],
        out_specs=pl.BlockSpec((1, 1, BQ, DH), lambda b, h, i: (b, h, i, 0)),
        out_shape=jax.ShapeDtypeStruct((B, H, S, DH), jnp.float32),
    )(q, k, v)

    stb = S // TB
    x = pl.pallas_call(
        _oproj_ln_body,
        grid=(NTB,),
        in_specs=[pl.BlockSpec((1, H, TB, DH),
                               lambda i: (i // stb, 0, i % stb, 0)),
                  pl.BlockSpec((D, D), lambda i: (0, 0)),
                  pl.BlockSpec((1, D), lambda i: (0, 0)),
                  pl.BlockSpec((TB, D), lambda i: (i, 0)),
                  pl.BlockSpec((1, D), lambda i: (0, 0)),
                  pl.BlockSpec((1, D), lambda i: (0, 0))],
        out_specs=pl.BlockSpec((TB, D), lambda i: (i, 0)),
        out_shape=jax.ShapeDtypeStruct((T, D), jnp.float32),
    )(o, Wo, bo.reshape(1, D), xf,
      ln1_g.reshape(1, D), ln1_b.reshape(1, D))

    y_sh = pl.pallas_call(
        _shared_body,
        grid=(NTB,),
        in_specs=[pl.BlockSpec((TB, D), lambda i: (i, 0)),
                  pl.BlockSpec((FFN, D), lambda i: (0, 0)),
                  pl.BlockSpec((1, FFN), lambda i: (0, 0)),
                  pl.BlockSpec((D, FFN), lambda i: (0, 0))],
        out_specs=pl.BlockSpec((TB, D), lambda i: (i, 0)),
        out_shape=jax.ShapeDtypeStruct((T, D), jnp.float32),
    )(x, fc1_W, fc1_b.reshape(1, FFN), fc2_W)

    gate_spec = pltpu.PrefetchScalarGridSpec(
        num_scalar_prefetch=1,
        grid=(B,),
        in_specs=[pl.BlockSpec((S, D), lambda b, idx: (b, 0)),
                  pl.BlockSpec((1, E, D), lambda b, idx: (idx[b], 0, 0)),
                  pl.BlockSpec((1, 1, E), lambda b, idx: (idx[b], 0, 0))],
        out_specs=[pl.BlockSpec((1, 1, S), lambda b, idx: (b, 0, 0)),
                   pl.BlockSpec((1, 1, S), lambda b, idx: (b, 0, 0))],
    )
    gate, gval = pl.pallas_call(
        _gate_body,
        grid_spec=gate_spec,
        out_shape=[jax.ShapeDtypeStruct((B, 1, S), jnp.int32),
                   jax.ShapeDtypeStruct((B, 1, S), jnp.float32)],
    )(idxes, x, gate_W, gate_b.reshape(ND, 1, E))

    gate_flat = gate.reshape(TROWS, 128)
    dst, bex, src = pl.pallas_call(
        _route_body,
        grid=(NB + 1,),
        in_specs=[pl.BlockSpec((TROWS, 128), lambda j: (0, 0))],
        out_specs=[pl.BlockSpec((TROWS, 128), lambda j: (0, 0)),
                   pl.BlockSpec((1, NB), lambda j: (0, 0)),
                   pl.BlockSpec((1, 1, TBS),
                                lambda j: (jnp.maximum(j - 1, 0), 0, 0))],
        out_shape=[jax.ShapeDtypeStruct((TROWS, 128), jnp.int32),
                   jax.ShapeDtypeStruct((1, NB), jnp.int32),
                   jax.ShapeDtypeStruct((NB, 1, TBS), jnp.int32)],
        scratch_shapes=[pltpu.VMEM((TROWS, 128), jnp.int32)],
    )(gate_flat)

    xs = _sc_gather(x, src.reshape(PAD_T), PAD_T)

    expert_spec = pltpu.PrefetchScalarGridSpec(
        num_scalar_prefetch=1,
        grid=(NB,),
        in_specs=[pl.BlockSpec((TBS, D), lambda i, bx: (i, 0)),
                  pl.BlockSpec((1, INTER, D), lambda i, bx: (bx[i], 0, 0)),
                  pl.BlockSpec((1, 1, INTER), lambda i, bx: (bx[i], 0, 0)),
                  pl.BlockSpec((1, D, INTER), lambda i, bx: (bx[i], 0, 0))],
        out_specs=pl.BlockSpec((TBS, D), lambda i, bx: (i, 0)),
    )
    ys = pl.pallas_call(
        _expert_body,
        grid_spec=expert_spec,
        out_shape=jax.ShapeDtypeStruct((PAD_T, D), jnp.float32),
    )(bex.reshape(NB), xs, exp1_W, exp1_b.reshape(E, 1, INTER), exp2_W)

    ye = _sc_gather(ys, dst.reshape(T), T)

    gval_r = gval.reshape(NTB, 1, TB)
    out = pl.pallas_call(
        _combine_body,
        grid=(NTB,),
        in_specs=[pl.BlockSpec((TB, D), lambda i: (i, 0)),
                  pl.BlockSpec((TB, D), lambda i: (i, 0)),
                  pl.BlockSpec((1, D), lambda i: (0, 0)),
                  pl.BlockSpec((1, 1, TB), lambda i: (i, 0, 0)),
                  pl.BlockSpec((TB, D), lambda i: (i, 0)),
                  pl.BlockSpec((1, D), lambda i: (0, 0)),
                  pl.BlockSpec((1, D), lambda i: (0, 0))],
        out_specs=pl.BlockSpec((TB, D), lambda i: (i, 0)),
        out_shape=jax.ShapeDtypeStruct((T, D), jnp.float32),
    )(y_sh, ye, fc2_b.reshape(1, D), gval_r, x,
      fln_g.reshape(1, D), fln_b.reshape(1, D))

    return out.reshape(B, S, D)


# speedup vs baseline: 1.4176x; 1.0018x over previous
import functools

import jax
import jax.numpy as jnp
from jax.experimental import pallas as pl
from jax.experimental.pallas import tpu as pltpu
from jax.experimental.pallas import tpu_sc as plsc

B, S, D, H = 2, 2048, 768, 12
FFN, INTER, E, ND = 3072, 768, 8, 4
DH = D // H
T = B * S
SCALE = DH ** -0.5

TB_QKV = 512
BQ = 1024
TB = 256
NTB = T // TB

TBS = 128
NB = 40
PAD_T = NB * TBS

TROWS = T // 128


def _gelu(x):
    return x * 0.5 * (1.0 + jax.lax.erf(x * (2.0 ** -0.5)))


def _layernorm(t, g, b):
    m = jnp.mean(t, axis=-1, keepdims=True)
    v = jnp.mean((t - m) ** 2, axis=-1, keepdims=True)
    return (t - m) / jnp.sqrt(v + 1e-5) * g + b


def _qkv_body(x_ref, wq_ref, wk_ref, wv_ref, b_ref, q_ref, k_ref, v_ref):
    x = x_ref[...]
    cn = (((1,), (1,)), ((), ()))
    q = (jax.lax.dot_general(x, wq_ref[...], cn,
                             preferred_element_type=jnp.float32)
         + b_ref[:, :D]) * SCALE
    k = (jax.lax.dot_general(x, wk_ref[...], cn,
                             preferred_element_type=jnp.float32)
         + b_ref[:, D:2 * D])
    v = (jax.lax.dot_general(x, wv_ref[...], cn,
                             preferred_element_type=jnp.float32)
         + b_ref[:, 2 * D:])
    q_ref[0] = jnp.stack([q[:, h * DH:(h + 1) * DH] for h in range(H)], 0)
    k_ref[0] = jnp.stack([k[:, h * DH:(h + 1) * DH] for h in range(H)], 0)
    v_ref[0] = jnp.stack([v[:, h * DH:(h + 1) * DH] for h in range(H)], 0)


def _attn_body(q_ref, k_ref, v_ref, o_ref):
    q = q_ref[0, 0]
    k = k_ref[0, 0]
    v = v_ref[0, 0]
    s = jax.lax.dot_general(q, k, (((1,), (1,)), ((), ())),
                            preferred_element_type=jnp.float32)
    m = jnp.max(s, axis=-1, keepdims=True)
    p = jnp.exp(s - m)
    p = p / jnp.sum(p, axis=-1, keepdims=True)
    o_ref[0, 0] = jnp.dot(p, v, preferred_element_type=jnp.float32)


def _oproj_ln_body(o_ref, w_ref, b_ref, hs_ref, g_ref, bb_ref, x_ref):
    o = jnp.concatenate([o_ref[0, h] for h in range(H)], axis=-1)
    t = (jax.lax.dot_general(o, w_ref[...], (((1,), (1,)), ((), ())),
                             preferred_element_type=jnp.float32)
         + b_ref[...] + hs_ref[...])
    x_ref[...] = _layernorm(t, g_ref[...], bb_ref[...])


def _gate_body(idx_ref, x_ref, gw_ref, gb_ref, gate_ref, gval_ref):
    del idx_ref
    x = x_ref[...]
    gw = gw_ref[0]
    logits = jax.lax.dot_general(x, gw, (((1,), (1,)), ((), ())),
                                 preferred_element_type=jnp.float32)
    logits = logits + gb_ref[0]
    m = jnp.max(logits, axis=-1, keepdims=True)
    p = jnp.exp(logits - m)
    sp = jnp.sum(p, axis=-1)
    gate_ref[0, 0] = jnp.argmax(logits, axis=-1).astype(jnp.int32)
    gval_ref[0, 0] = jnp.max(p, axis=-1) / sp


def _route_body(gate_ref, dst_ref, bex_ref, src_ref, dsts_ref):
    j = pl.program_id(0)

    @pl.when(j == 0)
    def _meta():
        g = gate_ref[...]
        l0 = jax.lax.broadcasted_iota(jnp.int32, (128, 128), 0)
        l1 = jax.lax.broadcasted_iota(jnp.int32, (128, 128), 1)
        slt_lane = (l0 < l1).astype(jnp.float32)
        r0 = jax.lax.broadcasted_iota(jnp.int32, (TROWS, TROWS), 0)
        r1 = jax.lax.broadcasted_iota(jnp.int32, (TROWS, TROWS), 1)
        slt_row = (r1 < r0).astype(jnp.float32)

        dst = jnp.zeros((TROWS, 128), jnp.float32)
        bpos = (jax.lax.broadcasted_iota(jnp.int32, (1, NB), 1)
                * TBS).astype(jnp.float32)
        bex = jnp.zeros((1, NB), jnp.float32)
        base = jnp.float32(0.0)
        for e in range(E):
            oh = (g == e).astype(jnp.float32)
            cs = jnp.dot(oh, slt_lane,
                         preferred_element_type=jnp.float32)
            rs = jnp.sum(oh, axis=1, keepdims=True)
            rp = jnp.dot(slt_row, rs,
                         preferred_element_type=jnp.float32)
            dst = dst + oh * (cs + rp + base)
            cnt = jnp.sum(oh)
            pc = jnp.ceil(cnt * (1.0 / TBS)) * TBS
            bex = bex + jnp.float32(e) * ((bpos >= base) &
                                          (bpos < base + pc)).astype(jnp.float32)
            base = base + pc
        dst_ref[...] = dst.astype(jnp.int32)
        dsts_ref[...] = dst.astype(jnp.int32)
        bex_ref[...] = bex.astype(jnp.int32)
        src_ref[0, 0] = jnp.zeros((TBS,), jnp.int32)

    @pl.when(j > 0)
    def _src():
        p0 = (j - 1) * TBS
        d = dsts_ref[...]
        pos = (jax.lax.broadcasted_iota(jnp.int32, (TBS, TROWS, 128), 0)
               + p0)
        eq = (d[None, :, :] == pos).astype(jnp.float32)
        tok = (jax.lax.broadcasted_iota(jnp.int32, (TBS, TROWS, 128), 1)
               * 128
               + jax.lax.broadcasted_iota(jnp.int32, (TBS, TROWS, 128), 2)
               ).astype(jnp.float32)
        s = jnp.sum(jnp.sum(eq * tok, axis=2), axis=1)
        hit = jnp.sum(jnp.sum(eq, axis=2), axis=1)
        pv = jax.lax.broadcasted_iota(jnp.int32, (TBS,), 0) + p0
        fb = (pv - T * (pv >= T)).astype(jnp.float32)
        src_ref[0, 0] = (s + (1.0 - hit) * fb).astype(jnp.int32)


def _expert_body(bex_ref, xs_ref, e1w_ref, e1b_ref, e2w_ref, ys_ref):
    del bex_ref
    xs = xs_ref[...].astype(jnp.bfloat16)
    he = jax.lax.dot_general(xs, e1w_ref[0], (((1,), (1,)), ((), ())),
                             preferred_element_type=jnp.float32)
    he = _gelu(he + e1b_ref[0]).astype(jnp.bfloat16)
    ys_ref[...] = jax.lax.dot_general(he, e2w_ref[0], (((1,), (1,)), ((), ())),
                                      preferred_element_type=jnp.float32)


def _shared_body(x_ref, w1_ref, b1_ref, w2_ref, o_ref):
    cn = (((1,), (1,)), ((), ()))
    h = jax.lax.dot_general(x_ref[...].astype(jnp.bfloat16), w1_ref[...], cn,
                            preferred_element_type=jnp.float32)
    h = _gelu(h + b1_ref[...]).astype(jnp.bfloat16)
    o_ref[...] = jax.lax.dot_general(h, w2_ref[...], cn,
                                     preferred_element_type=jnp.float32)


def _combine_body(ysh_ref, ye_ref, b2_ref, gval_ref, x_ref, g_ref, bb_ref,
                  o_ref):
    x = x_ref[...]
    y = ysh_ref[...] + ye_ref[...] + b2_ref[...]
    t = y * gval_ref[0, 0][:, None] + x
    o_ref[...] = _layernorm(t, g_ref[...], bb_ref[...])


def _sc_gather(table, idx, n_rows):
    info = plsc.get_sparse_core_info()
    nc, ns = info.num_cores, info.num_subcores
    nw = nc * ns
    per_w = n_rows // nw
    n_ch = 4
    ch = per_w // n_ch
    mesh = plsc.VectorSubcoreMesh(core_axis_name="c", subcore_axis_name="s")

    @functools.partial(
        pl.kernel, mesh=mesh,
        out_type=jax.ShapeDtypeStruct((n_rows, D), jnp.float32),
        scratch_types=[pltpu.VMEM((per_w,), jnp.int32)]
        + [pltpu.VMEM((ch, D), jnp.float32) for _ in range(4)]
        + [pltpu.SemaphoreType.DMA, pltpu.SemaphoreType.DMA])
    def gk(table_hbm, idx_hbm, out_hbm, idx_v, b0, b1, b2, b3, gsem, ssem):
        bufs = [b0, b1, b2, b3]
        wid = jax.lax.axis_index("s") * nc + jax.lax.axis_index("c")
        base = wid * per_w
        pltpu.sync_copy(idx_hbm.at[pl.ds(base, per_w)], idx_v)
        gathers = [pltpu.async_copy(
            table_hbm.at[idx_v.at[pl.ds(c * ch, ch)]], bufs[c], gsem)
            for c in range(n_ch)]
        stores = []
        for c in range(n_ch):
            gathers[c].wait()
            stores.append(pltpu.async_copy(
                bufs[c], out_hbm.at[pl.ds(base + c * ch, ch)], ssem))
        for cp in stores:
            cp.wait()

    return gk(table, idx)


def kernel(hidden_states, idxes, Wq, bq, Wk, bk, Wv, bv, Wo, bo, ln1_g, ln1_b,
           fc1_W, fc1_b, fc2_W, fc2_b, exp1_W, exp1_b, exp2_W, gate_W, gate_b,
           fln_g, fln_b):
    xf = hidden_states.reshape(T, D)
    bqkv = jnp.concatenate([bq, bk, bv]).reshape(1, 3 * D)
    sb = S // TB_QKV

    hspec = pl.BlockSpec((1, H, TB_QKV, DH), lambda i: (i // sb, 0, i % sb, 0))
    q, k, v = pl.pallas_call(
        _qkv_body,
        grid=(T // TB_QKV,),
        in_specs=[pl.BlockSpec((TB_QKV, D), lambda i: (i, 0)),
                  pl.BlockSpec((D, D), lambda i: (0, 0)),
                  pl.BlockSpec((D, D), lambda i: (0, 0)),
                  pl.BlockSpec((D, D), lambda i: (0, 0)),
                  pl.BlockSpec((1, 3 * D), lambda i: (0, 0))],
        out_specs=[hspec, hspec, hspec],
        out_shape=[jax.ShapeDtypeStruct((B, H, S, DH), jnp.float32)] * 3,
    )(xf, Wq, Wk, Wv, bqkv)

    o = pl.pallas_call(
        _attn_body,
        grid=(B, H, S // BQ),
        in_specs=[pl.BlockSpec((1, 1, BQ, DH), lambda b, h, i: (b, h, i, 0)),
                  pl.BlockSpec((1, 1, S, DH), lambda b, h, i: (b, h, 0, 0)),
                  pl.BlockSpec((1, 1, S, DH), lambda b, h, i: (b, h, 0, 0))],
        out_specs=pl.BlockSpec((1, 1, BQ, DH), lambda b, h, i: (b, h, i, 0)),
        out_shape=jax.ShapeDtypeStruct((B, H, S, DH), jnp.float32),
    )(q, k, v)

    stb = S // TB
    x = pl.pallas_call(
        _oproj_ln_body,
        grid=(NTB,),
        in_specs=[pl.BlockSpec((1, H, TB, DH),
                               lambda i: (i // stb, 0, i % stb, 0)),
                  pl.BlockSpec((D, D), lambda i: (0, 0)),
                  pl.BlockSpec((1, D), lambda i: (0, 0)),
                  pl.BlockSpec((TB, D), lambda i: (i, 0)),
                  pl.BlockSpec((1, D), lambda i: (0, 0)),
                  pl.BlockSpec((1, D), lambda i: (0, 0))],
        out_specs=pl.BlockSpec((TB, D), lambda i: (i, 0)),
        out_shape=jax.ShapeDtypeStruct((T, D), jnp.float32),
    )(o, Wo, bo.reshape(1, D), xf,
      ln1_g.reshape(1, D), ln1_b.reshape(1, D))

    y_sh = pl.pallas_call(
        _shared_body,
        grid=(NTB,),
        in_specs=[pl.BlockSpec((TB, D), lambda i: (i, 0)),
                  pl.BlockSpec((FFN, D), lambda i: (0, 0)),
                  pl.BlockSpec((1, FFN), lambda i: (0, 0)),
                  pl.BlockSpec((D, FFN), lambda i: (0, 0))],
        out_specs=pl.BlockSpec((TB, D), lambda i: (i, 0)),
        out_shape=jax.ShapeDtypeStruct((T, D), jnp.float32),
    )(x, fc1_W.astype(jnp.bfloat16), fc1_b.reshape(1, FFN),
      fc2_W.astype(jnp.bfloat16))

    gate_spec = pltpu.PrefetchScalarGridSpec(
        num_scalar_prefetch=1,
        grid=(B,),
        in_specs=[pl.BlockSpec((S, D), lambda b, idx: (b, 0)),
                  pl.BlockSpec((1, E, D), lambda b, idx: (idx[b], 0, 0)),
                  pl.BlockSpec((1, 1, E), lambda b, idx: (idx[b], 0, 0))],
        out_specs=[pl.BlockSpec((1, 1, S), lambda b, idx: (b, 0, 0)),
                   pl.BlockSpec((1, 1, S), lambda b, idx: (b, 0, 0))],
    )
    gate, gval = pl.pallas_call(
        _gate_body,
        grid_spec=gate_spec,
        out_shape=[jax.ShapeDtypeStruct((B, 1, S), jnp.int32),
                   jax.ShapeDtypeStruct((B, 1, S), jnp.float32)],
    )(idxes, x, gate_W, gate_b.reshape(ND, 1, E))

    gate_flat = gate.reshape(TROWS, 128)
    dst, bex, src = pl.pallas_call(
        _route_body,
        grid=(NB + 1,),
        in_specs=[pl.BlockSpec((TROWS, 128), lambda j: (0, 0))],
        out_specs=[pl.BlockSpec((TROWS, 128), lambda j: (0, 0)),
                   pl.BlockSpec((1, NB), lambda j: (0, 0)),
                   pl.BlockSpec((1, 1, TBS),
                                lambda j: (jnp.maximum(j - 1, 0), 0, 0))],
        out_shape=[jax.ShapeDtypeStruct((TROWS, 128), jnp.int32),
                   jax.ShapeDtypeStruct((1, NB), jnp.int32),
                   jax.ShapeDtypeStruct((NB, 1, TBS), jnp.int32)],
        scratch_shapes=[pltpu.VMEM((TROWS, 128), jnp.int32)],
    )(gate_flat)

    xs = _sc_gather(x, src.reshape(PAD_T), PAD_T)

    expert_spec = pltpu.PrefetchScalarGridSpec(
        num_scalar_prefetch=1,
        grid=(NB,),
        in_specs=[pl.BlockSpec((TBS, D), lambda i, bx: (i, 0)),
                  pl.BlockSpec((1, INTER, D), lambda i, bx: (bx[i], 0, 0)),
                  pl.BlockSpec((1, 1, INTER), lambda i, bx: (bx[i], 0, 0)),
                  pl.BlockSpec((1, D, INTER), lambda i, bx: (bx[i], 0, 0))],
        out_specs=pl.BlockSpec((TBS, D), lambda i, bx: (i, 0)),
    )
    ys = pl.pallas_call(
        _expert_body,
        grid_spec=expert_spec,
        out_shape=jax.ShapeDtypeStruct((PAD_T, D), jnp.float32),
    )(bex.reshape(NB), xs, exp1_W.astype(jnp.bfloat16),
      exp1_b.reshape(E, 1, INTER), exp2_W.astype(jnp.bfloat16))

    ye = _sc_gather(ys, dst.reshape(T), T)

    gval_r = gval.reshape(NTB, 1, TB)
    out = pl.pallas_call(
        _combine_body,
        grid=(NTB,),
        in_specs=[pl.BlockSpec((TB, D), lambda i: (i, 0)),
                  pl.BlockSpec((TB, D), lambda i: (i, 0)),
                  pl.BlockSpec((1, D), lambda i: (0, 0)),
                  pl.BlockSpec((1, 1, TB), lambda i: (i, 0, 0)),
                  pl.BlockSpec((TB, D), lambda i: (i, 0)),
                  pl.BlockSpec((1, D), lambda i: (0, 0)),
                  pl.BlockSpec((1, D), lambda i: (0, 0))],
        out_specs=pl.BlockSpec((TB, D), lambda i: (i, 0)),
        out_shape=jax.ShapeDtypeStruct((T, D), jnp.float32),
    )(y_sh, ye, fc2_b.reshape(1, D), gval_r, x,
      fln_g.reshape(1, D), fln_b.reshape(1, D))

    return out.reshape(B, S, D)


# speedup vs baseline: 1.5754x; 1.1113x over previous
import functools

import jax
import jax.numpy as jnp
from jax.experimental import pallas as pl
from jax.experimental.pallas import tpu as pltpu
from jax.experimental.pallas import tpu_sc as plsc

B, S, D, H = 2, 2048, 768, 12
FFN, INTER, E, ND = 3072, 768, 8, 4
DH = D // H
T = B * S
SCALE = DH ** -0.5

TB_QKV = 512
BQ = 1024
TB = 256
NTB = T // TB

TBS = 128
NB = 40
PAD_T = NB * TBS

TROWS = T // 128


def _gelu(x):
    return x * 0.5 * (1.0 + jax.lax.erf(x * (2.0 ** -0.5)))


def _layernorm(t, g, b):
    m = jnp.mean(t, axis=-1, keepdims=True)
    v = jnp.mean((t - m) ** 2, axis=-1, keepdims=True)
    return (t - m) / jnp.sqrt(v + 1e-5) * g + b


def _qkv_body(x_ref, wq_ref, wk_ref, wv_ref, b_ref, q_ref, k_ref, v_ref):
    x = x_ref[...]
    cn = (((1,), (1,)), ((), ()))
    q = (jax.lax.dot_general(x, wq_ref[...], cn,
                             preferred_element_type=jnp.float32)
         + b_ref[:, :D]) * SCALE
    k = (jax.lax.dot_general(x, wk_ref[...], cn,
                             preferred_element_type=jnp.float32)
         + b_ref[:, D:2 * D])
    v = (jax.lax.dot_general(x, wv_ref[...], cn,
                             preferred_element_type=jnp.float32)
         + b_ref[:, 2 * D:])
    q_ref[0] = jnp.stack([q[:, h * DH:(h + 1) * DH] for h in range(H)], 0)
    k_ref[0] = jnp.stack([k[:, h * DH:(h + 1) * DH] for h in range(H)], 0)
    v_ref[0] = jnp.stack([v[:, h * DH:(h + 1) * DH] for h in range(H)], 0)


def _attn_body(q_ref, k_ref, v_ref, o_ref):
    q = q_ref[0, 0]
    k = k_ref[0, 0]
    v = v_ref[0, 0]
    s = jax.lax.dot_general(q, k, (((1,), (1,)), ((), ())),
                            preferred_element_type=jnp.float32)
    m = jnp.max(s, axis=-1, keepdims=True)
    p = jnp.exp(s - m)
    p = p / jnp.sum(p, axis=-1, keepdims=True)
    o_ref[0, 0] = jnp.dot(p, v, preferred_element_type=jnp.float32)


def _oproj_ln_body(o_ref, w_ref, b_ref, hs_ref, g_ref, bb_ref, x_ref):
    o = jnp.concatenate([o_ref[0, h] for h in range(H)], axis=-1)
    t = (jax.lax.dot_general(o, w_ref[...], (((1,), (1,)), ((), ())),
                             preferred_element_type=jnp.float32)
         + b_ref[...] + hs_ref[...])
    x_ref[...] = _layernorm(t, g_ref[...], bb_ref[...])


def _gate_body(idx_ref, x_ref, gw_ref, gb_ref, gate_ref, gval_ref):
    del idx_ref
    x = x_ref[...]
    gw = gw_ref[0]
    logits = jax.lax.dot_general(x, gw, (((1,), (1,)), ((), ())),
                                 preferred_element_type=jnp.float32)
    logits = logits + gb_ref[0]
    m = jnp.max(logits, axis=-1, keepdims=True)
    p = jnp.exp(logits - m)
    sp = jnp.sum(p, axis=-1)
    gate_ref[0, 0] = jnp.argmax(logits, axis=-1).astype(jnp.int32)
    gval_ref[0, 0] = jnp.max(p, axis=-1) / sp


def _route_body(gate_ref, dst_ref, bex_ref):
    if True:
        g = gate_ref[...]
        l0 = jax.lax.broadcasted_iota(jnp.int32, (128, 128), 0)
        l1 = jax.lax.broadcasted_iota(jnp.int32, (128, 128), 1)
        slt_lane = (l0 < l1).astype(jnp.float32)
        r0 = jax.lax.broadcasted_iota(jnp.int32, (TROWS, TROWS), 0)
        r1 = jax.lax.broadcasted_iota(jnp.int32, (TROWS, TROWS), 1)
        slt_row = (r1 < r0).astype(jnp.float32)

        dst = jnp.zeros((TROWS, 128), jnp.float32)
        bpos = (jax.lax.broadcasted_iota(jnp.int32, (1, NB), 1)
                * TBS).astype(jnp.float32)
        bex = jnp.zeros((1, NB), jnp.float32)
        base = jnp.float32(0.0)
        for e in range(E):
            oh = (g == e).astype(jnp.float32)
            cs = jnp.dot(oh, slt_lane,
                         preferred_element_type=jnp.float32)
            rs = jnp.sum(oh, axis=1, keepdims=True)
            rp = jnp.dot(slt_row, rs,
                         preferred_element_type=jnp.float32)
            dst = dst + oh * (cs + rp + base)
            cnt = jnp.sum(oh)
            pc = jnp.ceil(cnt * (1.0 / TBS)) * TBS
            bex = bex + jnp.float32(e) * ((bpos >= base) &
                                          (bpos < base + pc)).astype(jnp.float32)
            base = base + pc
        dst_ref[...] = dst.astype(jnp.int32)
        bex_ref[...] = bex.astype(jnp.int32)


def _expert_body(bex_ref, xs_ref, e1w_ref, e1b_ref, e2w_ref, ys_ref):
    del bex_ref
    xs = xs_ref[...].astype(jnp.bfloat16)
    he = jax.lax.dot_general(xs, e1w_ref[0], (((1,), (1,)), ((), ())),
                             preferred_element_type=jnp.float32)
    he = _gelu(he + e1b_ref[0]).astype(jnp.bfloat16)
    ys_ref[...] = jax.lax.dot_general(he, e2w_ref[0], (((1,), (1,)), ((), ())),
                                      preferred_element_type=jnp.float32)


def _shared_body(x_ref, w1_ref, b1_ref, w2_ref, o_ref):
    cn = (((1,), (1,)), ((), ()))
    h = jax.lax.dot_general(x_ref[...].astype(jnp.bfloat16), w1_ref[...], cn,
                            preferred_element_type=jnp.float32)
    h = _gelu(h + b1_ref[...]).astype(jnp.bfloat16)
    o_ref[...] = jax.lax.dot_general(h, w2_ref[...], cn,
                                     preferred_element_type=jnp.float32)


def _combine_body(ysh_ref, ye_ref, b2_ref, gval_ref, x_ref, g_ref, bb_ref,
                  o_ref):
    x = x_ref[...]
    y = ysh_ref[...] + ye_ref[...] + b2_ref[...]
    t = y * gval_ref[0, 0][:, None] + x
    o_ref[...] = _layernorm(t, g_ref[...], bb_ref[...])


def _sc_scatter(table, idx, n_rows_out):
    info = plsc.get_sparse_core_info()
    nc, ns = info.num_cores, info.num_subcores
    nw = nc * ns
    n_rows_in = table.shape[0]
    per_w = n_rows_in // nw
    n_ch = 4
    ch = per_w // n_ch
    mesh = plsc.VectorSubcoreMesh(core_axis_name="c", subcore_axis_name="s")

    @functools.partial(
        pl.kernel, mesh=mesh,
        out_type=jax.ShapeDtypeStruct((n_rows_out, D), jnp.float32),
        scratch_types=[pltpu.VMEM((n_ch, ch), jnp.int32)]
        + [pltpu.VMEM((ch, D), jnp.float32) for _ in range(4)]
        + [pltpu.SemaphoreType.DMA, pltpu.SemaphoreType.DMA])
    def sk(table_hbm, idx_hbm, out_hbm, idx_v, b0, b1, b2, b3, lsem, ssem):
        bufs = [b0, b1, b2, b3]
        wid = jax.lax.axis_index("s") * nc + jax.lax.axis_index("c")
        base = wid * per_w
        loads = []
        for c in range(n_ch):
            pltpu.sync_copy(idx_hbm.at[pl.ds(base + c * ch, ch)], idx_v.at[c])
            loads.append(pltpu.async_copy(
                table_hbm.at[pl.ds(base + c * ch, ch)], bufs[c], lsem))
        stores = []
        for c in range(n_ch):
            loads[c].wait()
            stores.append(pltpu.async_copy(
                bufs[c], out_hbm.at[idx_v.at[c]], ssem))
        for cp in stores:
            cp.wait()

    return sk(table, idx)


def _sc_gather(table, idx, n_rows):
    info = plsc.get_sparse_core_info()
    nc, ns = info.num_cores, info.num_subcores
    nw = nc * ns
    per_w = n_rows // nw
    n_ch = 4
    ch = per_w // n_ch
    mesh = plsc.VectorSubcoreMesh(core_axis_name="c", subcore_axis_name="s")

    @functools.partial(
        pl.kernel, mesh=mesh,
        out_type=jax.ShapeDtypeStruct((n_rows, D), jnp.float32),
        scratch_types=[pltpu.VMEM((per_w,), jnp.int32)]
        + [pltpu.VMEM((ch, D), jnp.float32) for _ in range(4)]
        + [pltpu.SemaphoreType.DMA, pltpu.SemaphoreType.DMA])
    def gk(table_hbm, idx_hbm, out_hbm, idx_v, b0, b1, b2, b3, gsem, ssem):
        bufs = [b0, b1, b2, b3]
        wid = jax.lax.axis_index("s") * nc + jax.lax.axis_index("c")
        base = wid * per_w
        pltpu.sync_copy(idx_hbm.at[pl.ds(base, per_w)], idx_v)
        gathers = [pltpu.async_copy(
            table_hbm.at[idx_v.at[pl.ds(c * ch, ch)]], bufs[c], gsem)
            for c in range(n_ch)]
        stores = []
        for c in range(n_ch):
            gathers[c].wait()
            stores.append(pltpu.async_copy(
                bufs[c], out_hbm.at[pl.ds(base + c * ch, ch)], ssem))
        for cp in stores:
            cp.wait()

    return gk(table, idx)


def kernel(hidden_states, idxes, Wq, bq, Wk, bk, Wv, bv, Wo, bo, ln1_g, ln1_b,
           fc1_W, fc1_b, fc2_W, fc2_b, exp1_W, exp1_b, exp2_W, gate_W, gate_b,
           fln_g, fln_b):
    xf = hidden_states.reshape(T, D)
    bqkv = jnp.concatenate([bq, bk, bv]).reshape(1, 3 * D)
    sb = S // TB_QKV

    hspec = pl.BlockSpec((1, H, TB_QKV, DH), lambda i: (i // sb, 0, i % sb, 0))
    q, k, v = pl.pallas_call(
        _qkv_body,
        grid=(T // TB_QKV,),
        in_specs=[pl.BlockSpec((TB_QKV, D), lambda i: (i, 0)),
                  pl.BlockSpec((D, D), lambda i: (0, 0)),
                  pl.BlockSpec((D, D), lambda i: (0, 0)),
                  pl.BlockSpec((D, D), lambda i: (0, 0)),
                  pl.BlockSpec((1, 3 * D), lambda i: (0, 0))],
        out_specs=[hspec, hspec, hspec],
        out_shape=[jax.ShapeDtypeStruct((B, H, S, DH), jnp.float32)] * 3,
    )(xf, Wq, Wk, Wv, bqkv)

    o = pl.pallas_call(
        _attn_body,
        grid=(B, H, S // BQ),
        in_specs=[pl.BlockSpec((1, 1, BQ, DH), lambda b, h, i: (b, h, i, 0)),
                  pl.BlockSpec((1, 1, S, DH), lambda b, h, i: (b, h, 0, 0)),
                  pl.BlockSpec((1, 1, S, DH), lambda b, h, i: (b, h, 0, 0))],
        out_specs=pl.BlockSpec((1, 1, BQ, DH), lambda b, h, i: (b, h, i, 0)),
        out_shape=jax.ShapeDtypeStruct((B, H, S, DH), jnp.float32),
    )(q, k, v)

    stb = S // TB
    x = pl.pallas_call(
        _oproj_ln_body,
        grid=(NTB,),
        in_specs=[pl.BlockSpec((1, H, TB, DH),
                               lambda i: (i // stb, 0, i % stb, 0)),
                  pl.BlockSpec((D, D), lambda i: (0, 0)),
                  pl.BlockSpec((1, D), lambda i: (0, 0)),
                  pl.BlockSpec((TB, D), lambda i: (i, 0)),
                  pl.BlockSpec((1, D), lambda i: (0, 0)),
                  pl.BlockSpec((1, D), lambda i: (0, 0))],
        out_specs=pl.BlockSpec((TB, D), lambda i: (i, 0)),
        out_shape=jax.ShapeDtypeStruct((T, D), jnp.float32),
    )(o, Wo, bo.reshape(1, D), xf,
      ln1_g.reshape(1, D), ln1_b.reshape(1, D))

    y_sh = pl.pallas_call(
        _shared_body,
        grid=(NTB,),
        in_specs=[pl.BlockSpec((TB, D), lambda i: (i, 0)),
                  pl.BlockSpec((FFN, D), lambda i: (0, 0)),
                  pl.BlockSpec((1, FFN), lambda i: (0, 0)),
                  pl.BlockSpec((D, FFN), lambda i: (0, 0))],
        out_specs=pl.BlockSpec((TB, D), lambda i: (i, 0)),
        out_shape=jax.ShapeDtypeStruct((T, D), jnp.float32),
    )(x, fc1_W.astype(jnp.bfloat16), fc1_b.reshape(1, FFN),
      fc2_W.astype(jnp.bfloat16))

    gate_spec = pltpu.PrefetchScalarGridSpec(
        num_scalar_prefetch=1,
        grid=(B,),
        in_specs=[pl.BlockSpec((S, D), lambda b, idx: (b, 0)),
                  pl.BlockSpec((1, E, D), lambda b, idx: (idx[b], 0, 0)),
                  pl.BlockSpec((1, 1, E), lambda b, idx: (idx[b], 0, 0))],
        out_specs=[pl.BlockSpec((1, 1, S), lambda b, idx: (b, 0, 0)),
                   pl.BlockSpec((1, 1, S), lambda b, idx: (b, 0, 0))],
    )
    gate, gval = pl.pallas_call(
        _gate_body,
        grid_spec=gate_spec,
        out_shape=[jax.ShapeDtypeStruct((B, 1, S), jnp.int32),
                   jax.ShapeDtypeStruct((B, 1, S), jnp.float32)],
    )(idxes, x, gate_W, gate_b.reshape(ND, 1, E))

    gate_flat = gate.reshape(TROWS, 128)
    dst, bex = pl.pallas_call(
        _route_body,
        grid=(1,),
        in_specs=[pl.BlockSpec((TROWS, 128), lambda j: (0, 0))],
        out_specs=[pl.BlockSpec((TROWS, 128), lambda j: (0, 0)),
                   pl.BlockSpec((1, NB), lambda j: (0, 0))],
        out_shape=[jax.ShapeDtypeStruct((TROWS, 128), jnp.int32),
                   jax.ShapeDtypeStruct((1, NB), jnp.int32)],
    )(gate_flat)

    xs = _sc_scatter(x, dst.reshape(T), PAD_T)

    expert_spec = pltpu.PrefetchScalarGridSpec(
        num_scalar_prefetch=1,
        grid=(NB,),
        in_specs=[pl.BlockSpec((TBS, D), lambda i, bx: (i, 0)),
                  pl.BlockSpec((1, INTER, D), lambda i, bx: (bx[i], 0, 0)),
                  pl.BlockSpec((1, 1, INTER), lambda i, bx: (bx[i], 0, 0)),
                  pl.BlockSpec((1, D, INTER), lambda i, bx: (bx[i], 0, 0))],
        out_specs=pl.BlockSpec((TBS, D), lambda i, bx: (i, 0)),
    )
    ys = pl.pallas_call(
        _expert_body,
        grid_spec=expert_spec,
        out_shape=jax.ShapeDtypeStruct((PAD_T, D), jnp.float32),
    )(bex.reshape(NB), xs, exp1_W.astype(jnp.bfloat16),
      exp1_b.reshape(E, 1, INTER), exp2_W.astype(jnp.bfloat16))

    ye = _sc_gather(ys, dst.reshape(T), T)

    gval_r = gval.reshape(NTB, 1, TB)
    out = pl.pallas_call(
        _combine_body,
        grid=(NTB,),
        in_specs=[pl.BlockSpec((TB, D), lambda i: (i, 0)),
                  pl.BlockSpec((TB, D), lambda i: (i, 0)),
                  pl.BlockSpec((1, D), lambda i: (0, 0)),
                  pl.BlockSpec((1, 1, TB), lambda i: (i, 0, 0)),
                  pl.BlockSpec((TB, D), lambda i: (i, 0)),
                  pl.BlockSpec((1, D), lambda i: (0, 0)),
                  pl.BlockSpec((1, D), lambda i: (0, 0))],
        out_specs=pl.BlockSpec((TB, D), lambda i: (i, 0)),
        out_shape=jax.ShapeDtypeStruct((T, D), jnp.float32),
    )(y_sh, ye, fc2_b.reshape(1, D), gval_r, x,
      fln_g.reshape(1, D), fln_b.reshape(1, D))

    return out.reshape(B, S, D)


# speedup vs baseline: 1.6343x; 1.0374x over previous
import functools

import jax
import jax.numpy as jnp
from jax.experimental import pallas as pl
from jax.experimental.pallas import tpu as pltpu
from jax.experimental.pallas import tpu_sc as plsc

B, S, D, H = 2, 2048, 768, 12
FFN, INTER, E, ND = 3072, 768, 8, 4
DH = D // H
T = B * S
SCALE = DH ** -0.5

TB_QKV = 512
BQ = 1024
TB = 256
NTB = T // TB

TBS = 128
NB = 40
PAD_T = NB * TBS

TROWS = T // 128


def _gelu(x):
    return x * 0.5 * (1.0 + jax.lax.erf(x * (2.0 ** -0.5)))


def _layernorm(t, g, b):
    m = jnp.mean(t, axis=-1, keepdims=True)
    v = jnp.mean((t - m) ** 2, axis=-1, keepdims=True)
    return (t - m) / jnp.sqrt(v + 1e-5) * g + b


def _qkv_body(x_ref, wq_ref, wk_ref, wv_ref, b_ref, q_ref, k_ref, v_ref):
    x = x_ref[...]
    cn = (((1,), (1,)), ((), ()))
    q = (jax.lax.dot_general(x, wq_ref[...], cn,
                             preferred_element_type=jnp.float32)
         + b_ref[:, :D]) * SCALE
    k = (jax.lax.dot_general(x, wk_ref[...], cn,
                             preferred_element_type=jnp.float32)
         + b_ref[:, D:2 * D])
    v = (jax.lax.dot_general(x, wv_ref[...], cn,
                             preferred_element_type=jnp.float32)
         + b_ref[:, 2 * D:])
    q_ref[0] = jnp.stack([q[:, h * DH:(h + 1) * DH] for h in range(H)], 0)
    k_ref[0] = jnp.stack([k[:, h * DH:(h + 1) * DH] for h in range(H)], 0)
    v_ref[0] = jnp.stack([v[:, h * DH:(h + 1) * DH] for h in range(H)], 0)


def _attn_body(q_ref, k_ref, v_ref, o_ref):
    q = q_ref[0, 0]
    k = k_ref[0, 0]
    v = v_ref[0, 0]
    s = jax.lax.dot_general(q, k, (((1,), (1,)), ((), ())),
                            preferred_element_type=jnp.float32)
    m = jnp.max(s, axis=-1, keepdims=True)
    p = jnp.exp(s - m)
    p = p / jnp.sum(p, axis=-1, keepdims=True)
    o_ref[0, 0] = jnp.dot(p, v, preferred_element_type=jnp.float32)


def _oproj_ln_body(o_ref, w_ref, b_ref, hs_ref, g_ref, bb_ref, x_ref):
    o = jnp.concatenate([o_ref[0, h] for h in range(H)], axis=-1)
    t = (jax.lax.dot_general(o, w_ref[...], (((1,), (1,)), ((), ())),
                             preferred_element_type=jnp.float32)
         + b_ref[...] + hs_ref[...])
    x_ref[...] = _layernorm(t, g_ref[...], bb_ref[...])


def _gate_body(idx_ref, x_ref, gw_ref, gb_ref, gate_ref, gval_ref):
    del idx_ref
    x = x_ref[...]
    gw = gw_ref[0]
    logits = jax.lax.dot_general(x, gw, (((1,), (1,)), ((), ())),
                                 preferred_element_type=jnp.float32)
    logits = logits + gb_ref[0]
    m = jnp.max(logits, axis=-1, keepdims=True)
    p = jnp.exp(logits - m)
    sp = jnp.sum(p, axis=-1)
    gate_ref[0, 0] = jnp.argmax(logits, axis=-1).astype(jnp.int32)
    gval_ref[0, 0] = jnp.max(p, axis=-1) / sp


def _route_body(gate_ref, dst_ref, bex_ref):
    if True:
        g = gate_ref[...]
        l0 = jax.lax.broadcasted_iota(jnp.int32, (128, 128), 0)
        l1 = jax.lax.broadcasted_iota(jnp.int32, (128, 128), 1)
        slt_lane = (l0 < l1).astype(jnp.float32)
        r0 = jax.lax.broadcasted_iota(jnp.int32, (TROWS, TROWS), 0)
        r1 = jax.lax.broadcasted_iota(jnp.int32, (TROWS, TROWS), 1)
        slt_row = (r1 < r0).astype(jnp.float32)

        dst = jnp.zeros((TROWS, 128), jnp.float32)
        bpos = (jax.lax.broadcasted_iota(jnp.int32, (1, NB), 1)
                * TBS).astype(jnp.float32)
        bex = jnp.zeros((1, NB), jnp.float32)
        base = jnp.float32(0.0)
        for e in range(E):
            oh = (g == e).astype(jnp.float32)
            cs = jnp.dot(oh, slt_lane,
                         preferred_element_type=jnp.float32)
            rs = jnp.sum(oh, axis=1, keepdims=True)
            rp = jnp.dot(slt_row, rs,
                         preferred_element_type=jnp.float32)
            dst = dst + oh * (cs + rp + base)
            cnt = jnp.sum(oh)
            pc = jnp.ceil(cnt * (1.0 / TBS)) * TBS
            bex = bex + jnp.float32(e) * ((bpos >= base) &
                                          (bpos < base + pc)).astype(jnp.float32)
            base = base + pc
        dst_ref[...] = dst.astype(jnp.int32)
        bex_ref[...] = bex.astype(jnp.int32)


def _expert_body(bex_ref, xs_ref, e1w_ref, e1b_ref, e2w_ref, ys_ref):
    del bex_ref
    xs = xs_ref[...]
    he = jax.lax.dot_general(xs, e1w_ref[0], (((1,), (1,)), ((), ())),
                             preferred_element_type=jnp.float32)
    he = _gelu(he + e1b_ref[0])
    ys_ref[...] = jax.lax.dot_general(he, e2w_ref[0], (((1,), (1,)), ((), ())),
                                      preferred_element_type=jnp.float32)


def _shared_body(x_ref, w1_ref, b1_ref, w2_ref, o_ref):
    cn = (((1,), (1,)), ((), ()))
    h = jax.lax.dot_general(x_ref[...], w1_ref[...], cn,
                            preferred_element_type=jnp.float32)
    h = _gelu(h + b1_ref[...])
    o_ref[...] = jax.lax.dot_general(h, w2_ref[...], cn,
                                     preferred_element_type=jnp.float32)


def _combine_body(ysh_ref, ye_ref, b2_ref, gval_ref, x_ref, g_ref, bb_ref,
                  o_ref):
    x = x_ref[...]
    y = ysh_ref[...] + ye_ref[...] + b2_ref[...]
    t = y * gval_ref[0, 0][:, None] + x
    o_ref[...] = _layernorm(t, g_ref[...], bb_ref[...])


def _sc_scatter(table, idx, n_rows_out):
    info = plsc.get_sparse_core_info()
    nc, ns = info.num_cores, info.num_subcores
    nw = nc * ns
    n_rows_in = table.shape[0]
    per_w = n_rows_in // nw
    n_ch = 4
    ch = per_w // n_ch
    mesh = plsc.VectorSubcoreMesh(core_axis_name="c", subcore_axis_name="s")

    @functools.partial(
        pl.kernel, mesh=mesh,
        out_type=jax.ShapeDtypeStruct((n_rows_out, D), jnp.float32),
        scratch_types=[pltpu.VMEM((n_ch, ch), jnp.int32)]
        + [pltpu.VMEM((ch, D), jnp.float32) for _ in range(4)]
        + [pltpu.SemaphoreType.DMA, pltpu.SemaphoreType.DMA])
    def sk(table_hbm, idx_hbm, out_hbm, idx_v, b0, b1, b2, b3, lsem, ssem):
        bufs = [b0, b1, b2, b3]
        wid = jax.lax.axis_index("s") * nc + jax.lax.axis_index("c")
        base = wid * per_w
        loads = []
        for c in range(n_ch):
            pltpu.sync_copy(idx_hbm.at[pl.ds(base + c * ch, ch)], idx_v.at[c])
            loads.append(pltpu.async_copy(
                table_hbm.at[pl.ds(base + c * ch, ch)], bufs[c], lsem))
        stores = []
        for c in range(n_ch):
            loads[c].wait()
            stores.append(pltpu.async_copy(
                bufs[c], out_hbm.at[idx_v.at[c]], ssem))
        for cp in stores:
            cp.wait()

    return sk(table, idx)


def _sc_gather(table, idx, n_rows):
    info = plsc.get_sparse_core_info()
    nc, ns = info.num_cores, info.num_subcores
    nw = nc * ns
    per_w = n_rows // nw
    n_ch = 4
    ch = per_w // n_ch
    mesh = plsc.VectorSubcoreMesh(core_axis_name="c", subcore_axis_name="s")

    @functools.partial(
        pl.kernel, mesh=mesh,
        out_type=jax.ShapeDtypeStruct((n_rows, D), jnp.float32),
        scratch_types=[pltpu.VMEM((per_w,), jnp.int32)]
        + [pltpu.VMEM((ch, D), jnp.float32) for _ in range(4)]
        + [pltpu.SemaphoreType.DMA, pltpu.SemaphoreType.DMA])
    def gk(table_hbm, idx_hbm, out_hbm, idx_v, b0, b1, b2, b3, gsem, ssem):
        bufs = [b0, b1, b2, b3]
        wid = jax.lax.axis_index("s") * nc + jax.lax.axis_index("c")
        base = wid * per_w
        pltpu.sync_copy(idx_hbm.at[pl.ds(base, per_w)], idx_v)
        gathers = [pltpu.async_copy(
            table_hbm.at[idx_v.at[pl.ds(c * ch, ch)]], bufs[c], gsem)
            for c in range(n_ch)]
        stores = []
        for c in range(n_ch):
            gathers[c].wait()
            stores.append(pltpu.async_copy(
                bufs[c], out_hbm.at[pl.ds(base + c * ch, ch)], ssem))
        for cp in stores:
            cp.wait()

    return gk(table, idx)


def kernel(hidden_states, idxes, Wq, bq, Wk, bk, Wv, bv, Wo, bo, ln1_g, ln1_b,
           fc1_W, fc1_b, fc2_W, fc2_b, exp1_W, exp1_b, exp2_W, gate_W, gate_b,
           fln_g, fln_b):
    xf = hidden_states.reshape(T, D)
    bqkv = jnp.concatenate([bq, bk, bv]).reshape(1, 3 * D)
    sb = S // TB_QKV

    hspec = pl.BlockSpec((1, H, TB_QKV, DH), lambda i: (i // sb, 0, i % sb, 0))
    q, k, v = pl.pallas_call(
        _qkv_body,
        grid=(T // TB_QKV,),
        in_specs=[pl.BlockSpec((TB_QKV, D), lambda i: (i, 0)),
                  pl.BlockSpec((D, D), lambda i: (0, 0)),
                  pl.BlockSpec((D, D), lambda i: (0, 0)),
                  pl.BlockSpec((D, D), lambda i: (0, 0)),
                  pl.BlockSpec((1, 3 * D), lambda i: (0, 0))],
        out_specs=[hspec, hspec, hspec],
        out_shape=[jax.ShapeDtypeStruct((B, H, S, DH), jnp.float32)] * 3,
    )(xf, Wq, Wk, Wv, bqkv)

    o = pl.pallas_call(
        _attn_body,
        grid=(B, H, S // BQ),
        in_specs=[pl.BlockSpec((1, 1, BQ, DH), lambda b, h, i: (b, h, i, 0)),
                  pl.BlockSpec((1, 1, S, DH), lambda b, h, i: (b, h, 0, 0)),
                  pl.BlockSpec((1, 1, S, DH), lambda b, h, i: (b, h, 0, 0))],
        out_specs=pl.BlockSpec((1, 1, BQ, DH), lambda b, h, i: (b, h, i, 0)),
        out_shape=jax.ShapeDtypeStruct((B, H, S, DH), jnp.float32),
    )(q, k, v)

    stb = S // TB
    x = pl.pallas_call(
        _oproj_ln_body,
        grid=(NTB,),
        in_specs=[pl.BlockSpec((1, H, TB, DH),
                               lambda i: (i // stb, 0, i % stb, 0)),
                  pl.BlockSpec((D, D), lambda i: (0, 0)),
                  pl.BlockSpec((1, D), lambda i: (0, 0)),
                  pl.BlockSpec((TB, D), lambda i: (i, 0)),
                  pl.BlockSpec((1, D), lambda i: (0, 0)),
                  pl.BlockSpec((1, D), lambda i: (0, 0))],
        out_specs=pl.BlockSpec((TB, D), lambda i: (i, 0)),
        out_shape=jax.ShapeDtypeStruct((T, D), jnp.float32),
    )(o, Wo, bo.reshape(1, D), xf,
      ln1_g.reshape(1, D), ln1_b.reshape(1, D))

    y_sh = pl.pallas_call(
        _shared_body,
        grid=(NTB,),
        in_specs=[pl.BlockSpec((TB, D), lambda i: (i, 0)),
                  pl.BlockSpec((FFN, D), lambda i: (0, 0)),
                  pl.BlockSpec((1, FFN), lambda i: (0, 0)),
                  pl.BlockSpec((D, FFN), lambda i: (0, 0))],
        out_specs=pl.BlockSpec((TB, D), lambda i: (i, 0)),
        out_shape=jax.ShapeDtypeStruct((T, D), jnp.float32),
    )(x, fc1_W, fc1_b.reshape(1, FFN), fc2_W)

    gate_spec = pltpu.PrefetchScalarGridSpec(
        num_scalar_prefetch=1,
        grid=(B,),
        in_specs=[pl.BlockSpec((S, D), lambda b, idx: (b, 0)),
                  pl.BlockSpec((1, E, D), lambda b, idx: (idx[b], 0, 0)),
                  pl.BlockSpec((1, 1, E), lambda b, idx: (idx[b], 0, 0))],
        out_specs=[pl.BlockSpec((1, 1, S), lambda b, idx: (b, 0, 0)),
                   pl.BlockSpec((1, 1, S), lambda b, idx: (b, 0, 0))],
    )
    gate, gval = pl.pallas_call(
        _gate_body,
        grid_spec=gate_spec,
        out_shape=[jax.ShapeDtypeStruct((B, 1, S), jnp.int32),
                   jax.ShapeDtypeStruct((B, 1, S), jnp.float32)],
    )(idxes, x, gate_W, gate_b.reshape(ND, 1, E))

    gate_flat = gate.reshape(TROWS, 128)
    dst, bex = pl.pallas_call(
        _route_body,
        grid=(1,),
        in_specs=[pl.BlockSpec((TROWS, 128), lambda j: (0, 0))],
        out_specs=[pl.BlockSpec((TROWS, 128), lambda j: (0, 0)),
                   pl.BlockSpec((1, NB), lambda j: (0, 0))],
        out_shape=[jax.ShapeDtypeStruct((TROWS, 128), jnp.int32),
                   jax.ShapeDtypeStruct((1, NB), jnp.int32)],
    )(gate_flat)

    xs = _sc_scatter(x, dst.reshape(T), PAD_T)

    expert_spec = pltpu.PrefetchScalarGridSpec(
        num_scalar_prefetch=1,
        grid=(NB,),
        in_specs=[pl.BlockSpec((TBS, D), lambda i, bx: (i, 0)),
                  pl.BlockSpec((1, INTER, D), lambda i, bx: (bx[i], 0, 0)),
                  pl.BlockSpec((1, 1, INTER), lambda i, bx: (bx[i], 0, 0)),
                  pl.BlockSpec((1, D, INTER), lambda i, bx: (bx[i], 0, 0))],
        out_specs=pl.BlockSpec((TBS, D), lambda i, bx: (i, 0)),
    )
    ys = pl.pallas_call(
        _expert_body,
        grid_spec=expert_spec,
        out_shape=jax.ShapeDtypeStruct((PAD_T, D), jnp.float32),
    )(bex.reshape(NB), xs, exp1_W,
      exp1_b.reshape(E, 1, INTER), exp2_W)

    ye = _sc_gather(ys, dst.reshape(T), T)

    gval_r = gval.reshape(NTB, 1, TB)
    out = pl.pallas_call(
        _combine_body,
        grid=(NTB,),
        in_specs=[pl.BlockSpec((TB, D), lambda i: (i, 0)),
                  pl.BlockSpec((TB, D), lambda i: (i, 0)),
                  pl.BlockSpec((1, D), lambda i: (0, 0)),
                  pl.BlockSpec((1, 1, TB), lambda i: (i, 0, 0)),
                  pl.BlockSpec((TB, D), lambda i: (i, 0)),
                  pl.BlockSpec((1, D), lambda i: (0, 0)),
                  pl.BlockSpec((1, D), lambda i: (0, 0))],
        out_specs=pl.BlockSpec((TB, D), lambda i: (i, 0)),
        out_shape=jax.ShapeDtypeStruct((T, D), jnp.float32),
    )(y_sh, ye, fc2_b.reshape(1, D), gval_r, x,
      fln_g.reshape(1, D), fln_b.reshape(1, D))

    return out.reshape(B, S, D)


# speedup vs baseline: 1.6692x; 1.0214x over previous
import functools

import jax
import jax.numpy as jnp
from jax.experimental import pallas as pl
from jax.experimental.pallas import tpu as pltpu
from jax.experimental.pallas import tpu_sc as plsc

B, S, D, H = 2, 2048, 768, 12
FFN, INTER, E, ND = 3072, 768, 8, 4
DH = D // H
T = B * S
SCALE = DH ** -0.5

TB_QKV = 512
BQ = 2048
TB = 256
NTB = T // TB

TBS = 128
NB = 40
PAD_T = NB * TBS

TROWS = T // 128


def _gelu(x):
    return x * 0.5 * (1.0 + jax.lax.erf(x * (2.0 ** -0.5)))


def _layernorm(t, g, b):
    m = jnp.mean(t, axis=-1, keepdims=True)
    v = jnp.mean((t - m) ** 2, axis=-1, keepdims=True)
    return (t - m) / jnp.sqrt(v + 1e-5) * g + b


def _qkv_body(x_ref, wq_ref, wk_ref, wv_ref, b_ref, q_ref, k_ref, v_ref):
    x = x_ref[...]
    cn = (((1,), (1,)), ((), ()))
    q = (jax.lax.dot_general(x, wq_ref[...], cn,
                             preferred_element_type=jnp.float32)
         + b_ref[:, :D]) * SCALE
    k = (jax.lax.dot_general(x, wk_ref[...], cn,
                             preferred_element_type=jnp.float32)
         + b_ref[:, D:2 * D])
    v = (jax.lax.dot_general(x, wv_ref[...], cn,
                             preferred_element_type=jnp.float32)
         + b_ref[:, 2 * D:])
    q_ref[0] = jnp.stack([q[:, h * DH:(h + 1) * DH] for h in range(H)], 0)
    k_ref[0] = jnp.stack([k[:, h * DH:(h + 1) * DH] for h in range(H)], 0)
    v_ref[0] = jnp.stack([v[:, h * DH:(h + 1) * DH] for h in range(H)], 0)


def _attn_body(q_ref, k_ref, v_ref, o_ref):
    q = q_ref[0, 0]
    k = k_ref[0, 0]
    v = v_ref[0, 0]
    s = jax.lax.dot_general(q, k, (((1,), (1,)), ((), ())),
                            preferred_element_type=jnp.float32)
    m = jnp.max(s, axis=-1, keepdims=True)
    p = jnp.exp(s - m)
    p = p / jnp.sum(p, axis=-1, keepdims=True)
    o_ref[0, 0] = jnp.dot(p, v, preferred_element_type=jnp.float32)


def _oproj_ln_body(o_ref, w_ref, b_ref, hs_ref, g_ref, bb_ref, x_ref):
    o = jnp.concatenate([o_ref[0, h] for h in range(H)], axis=-1)
    t = (jax.lax.dot_general(o, w_ref[...], (((1,), (1,)), ((), ())),
                             preferred_element_type=jnp.float32)
         + b_ref[...] + hs_ref[...])
    x_ref[...] = _layernorm(t, g_ref[...], bb_ref[...])


def _gate_body(idx_ref, x_ref, gw_ref, gb_ref, gate_ref, gval_ref):
    del idx_ref
    x = x_ref[...]
    gw = gw_ref[0]
    logits = jax.lax.dot_general(x, gw, (((1,), (1,)), ((), ())),
                                 preferred_element_type=jnp.float32)
    logits = logits + gb_ref[0]
    m = jnp.max(logits, axis=-1, keepdims=True)
    p = jnp.exp(logits - m)
    sp = jnp.sum(p, axis=-1)
    gate_ref[0, 0] = jnp.argmax(logits, axis=-1).astype(jnp.int32)
    gval_ref[0, 0] = jnp.max(p, axis=-1) / sp


def _route_body(gate_ref, dst_ref, bex_ref):
    if True:
        g = gate_ref[...]
        l0 = jax.lax.broadcasted_iota(jnp.int32, (128, 128), 0)
        l1 = jax.lax.broadcasted_iota(jnp.int32, (128, 128), 1)
        slt_lane = (l0 < l1).astype(jnp.float32)
        r0 = jax.lax.broadcasted_iota(jnp.int32, (TROWS, TROWS), 0)
        r1 = jax.lax.broadcasted_iota(jnp.int32, (TROWS, TROWS), 1)
        slt_row = (r1 < r0).astype(jnp.float32)

        dst = jnp.zeros((TROWS, 128), jnp.float32)
        bpos = (jax.lax.broadcasted_iota(jnp.int32, (1, NB), 1)
                * TBS).astype(jnp.float32)
        bex = jnp.zeros((1, NB), jnp.float32)
        base = jnp.float32(0.0)
        for e in range(E):
            oh = (g == e).astype(jnp.float32)
            cs = jnp.dot(oh, slt_lane,
                         preferred_element_type=jnp.float32)
            rs = jnp.sum(oh, axis=1, keepdims=True)
            rp = jnp.dot(slt_row, rs,
                         preferred_element_type=jnp.float32)
            dst = dst + oh * (cs + rp + base)
            cnt = jnp.sum(oh)
            pc = jnp.ceil(cnt * (1.0 / TBS)) * TBS
            bex = bex + jnp.float32(e) * ((bpos >= base) &
                                          (bpos < base + pc)).astype(jnp.float32)
            base = base + pc
        dst_ref[...] = dst.astype(jnp.int32)
        bex_ref[...] = bex.astype(jnp.int32)


def _expert_body(bex_ref, xs_ref, e1w_ref, e1b_ref, e2w_ref, ys_ref):
    del bex_ref
    xs = xs_ref[...]
    he = jax.lax.dot_general(xs, e1w_ref[0], (((1,), (1,)), ((), ())),
                             preferred_element_type=jnp.float32)
    he = _gelu(he + e1b_ref[0])
    ys_ref[...] = jax.lax.dot_general(he, e2w_ref[0], (((1,), (1,)), ((), ())),
                                      preferred_element_type=jnp.float32)


def _shared_body(x_ref, w1_ref, b1_ref, w2_ref, o_ref):
    cn = (((1,), (1,)), ((), ()))
    h = jax.lax.dot_general(x_ref[...], w1_ref[...], cn,
                            preferred_element_type=jnp.float32)
    h = _gelu(h + b1_ref[...])
    o_ref[...] = jax.lax.dot_general(h, w2_ref[...], cn,
                                     preferred_element_type=jnp.float32)


def _combine_body(ysh_ref, ye_ref, b2_ref, gval_ref, x_ref, g_ref, bb_ref,
                  o_ref):
    x = x_ref[...]
    y = ysh_ref[...] + ye_ref[...] + b2_ref[...]
    t = y * gval_ref[0, 0][:, None] + x
    o_ref[...] = _layernorm(t, g_ref[...], bb_ref[...])


def _sc_scatter(table, idx, n_rows_out):
    info = plsc.get_sparse_core_info()
    nc, ns = info.num_cores, info.num_subcores
    nw = nc * ns
    n_rows_in = table.shape[0]
    per_w = n_rows_in // nw
    n_ch = 4
    ch = per_w // n_ch
    mesh = plsc.VectorSubcoreMesh(core_axis_name="c", subcore_axis_name="s")

    @functools.partial(
        pl.kernel, mesh=mesh,
        out_type=jax.ShapeDtypeStruct((n_rows_out, D), jnp.float32),
        scratch_types=[pltpu.VMEM((n_ch, ch), jnp.int32)]
        + [pltpu.VMEM((ch, D), jnp.float32) for _ in range(4)]
        + [pltpu.SemaphoreType.DMA, pltpu.SemaphoreType.DMA])
    def sk(table_hbm, idx_hbm, out_hbm, idx_v, b0, b1, b2, b3, lsem, ssem):
        bufs = [b0, b1, b2, b3]
        wid = jax.lax.axis_index("s") * nc + jax.lax.axis_index("c")
        base = wid * per_w
        loads = []
        for c in range(n_ch):
            pltpu.sync_copy(idx_hbm.at[pl.ds(base + c * ch, ch)], idx_v.at[c])
            loads.append(pltpu.async_copy(
                table_hbm.at[pl.ds(base + c * ch, ch)], bufs[c], lsem))
        stores = []
        for c in range(n_ch):
            loads[c].wait()
            stores.append(pltpu.async_copy(
                bufs[c], out_hbm.at[idx_v.at[c]], ssem))
        for cp in stores:
            cp.wait()

    return sk(table, idx)


def _sc_gather(table, idx, n_rows):
    info = plsc.get_sparse_core_info()
    nc, ns = info.num_cores, info.num_subcores
    nw = nc * ns
    per_w = n_rows // nw
    n_ch = 4
    ch = per_w // n_ch
    mesh = plsc.VectorSubcoreMesh(core_axis_name="c", subcore_axis_name="s")

    @functools.partial(
        pl.kernel, mesh=mesh,
        out_type=jax.ShapeDtypeStruct((n_rows, D), jnp.float32),
        scratch_types=[pltpu.VMEM((per_w,), jnp.int32)]
        + [pltpu.VMEM((ch, D), jnp.float32) for _ in range(4)]
        + [pltpu.SemaphoreType.DMA, pltpu.SemaphoreType.DMA])
    def gk(table_hbm, idx_hbm, out_hbm, idx_v, b0, b1, b2, b3, gsem, ssem):
        bufs = [b0, b1, b2, b3]
        wid = jax.lax.axis_index("s") * nc + jax.lax.axis_index("c")
        base = wid * per_w
        pltpu.sync_copy(idx_hbm.at[pl.ds(base, per_w)], idx_v)
        gathers = [pltpu.async_copy(
            table_hbm.at[idx_v.at[pl.ds(c * ch, ch)]], bufs[c], gsem)
            for c in range(n_ch)]
        stores = []
        for c in range(n_ch):
            gathers[c].wait()
            stores.append(pltpu.async_copy(
                bufs[c], out_hbm.at[pl.ds(base + c * ch, ch)], ssem))
        for cp in stores:
            cp.wait()

    return gk(table, idx)


def kernel(hidden_states, idxes, Wq, bq, Wk, bk, Wv, bv, Wo, bo, ln1_g, ln1_b,
           fc1_W, fc1_b, fc2_W, fc2_b, exp1_W, exp1_b, exp2_W, gate_W, gate_b,
           fln_g, fln_b):
    xf = hidden_states.reshape(T, D)
    bqkv = jnp.concatenate([bq, bk, bv]).reshape(1, 3 * D)
    sb = S // TB_QKV

    hspec = pl.BlockSpec((1, H, TB_QKV, DH), lambda i: (i // sb, 0, i % sb, 0))
    q, k, v = pl.pallas_call(
        _qkv_body,
        grid=(T // TB_QKV,),
        in_specs=[pl.BlockSpec((TB_QKV, D), lambda i: (i, 0)),
                  pl.BlockSpec((D, D), lambda i: (0, 0)),
                  pl.BlockSpec((D, D), lambda i: (0, 0)),
                  pl.BlockSpec((D, D), lambda i: (0, 0)),
                  pl.BlockSpec((1, 3 * D), lambda i: (0, 0))],
        out_specs=[hspec, hspec, hspec],
        out_shape=[jax.ShapeDtypeStruct((B, H, S, DH), jnp.float32)] * 3,
    )(xf, Wq, Wk, Wv, bqkv)

    o = pl.pallas_call(
        _attn_body,
        grid=(B, H, S // BQ),
        in_specs=[pl.BlockSpec((1, 1, BQ, DH), lambda b, h, i: (b, h, i, 0)),
                  pl.BlockSpec((1, 1, S, DH), lambda b, h, i: (b, h, 0, 0)),
                  pl.BlockSpec((1, 1, S, DH), lambda b, h, i: (b, h, 0, 0))],
        out_specs=pl.BlockSpec((1, 1, BQ, DH), lambda b, h, i: (b, h, i, 0)),
        out_shape=jax.ShapeDtypeStruct((B, H, S, DH), jnp.float32),
    )(q, k, v)

    stb = S // TB
    x = pl.pallas_call(
        _oproj_ln_body,
        grid=(NTB,),
        in_specs=[pl.BlockSpec((1, H, TB, DH),
                               lambda i: (i // stb, 0, i % stb, 0)),
                  pl.BlockSpec((D, D), lambda i: (0, 0)),
                  pl.BlockSpec((1, D), lambda i: (0, 0)),
                  pl.BlockSpec((TB, D), lambda i: (i, 0)),
                  pl.BlockSpec((1, D), lambda i: (0, 0)),
                  pl.BlockSpec((1, D), lambda i: (0, 0))],
        out_specs=pl.BlockSpec((TB, D), lambda i: (i, 0)),
        out_shape=jax.ShapeDtypeStruct((T, D), jnp.float32),
    )(o, Wo, bo.reshape(1, D), xf,
      ln1_g.reshape(1, D), ln1_b.reshape(1, D))

    y_sh = pl.pallas_call(
        _shared_body,
        grid=(NTB,),
        in_specs=[pl.BlockSpec((TB, D), lambda i: (i, 0)),
                  pl.BlockSpec((FFN, D), lambda i: (0, 0)),
                  pl.BlockSpec((1, FFN), lambda i: (0, 0)),
                  pl.BlockSpec((D, FFN), lambda i: (0, 0))],
        out_specs=pl.BlockSpec((TB, D), lambda i: (i, 0)),
        out_shape=jax.ShapeDtypeStruct((T, D), jnp.float32),
    )(x, fc1_W, fc1_b.reshape(1, FFN), fc2_W)

    gate_spec = pltpu.PrefetchScalarGridSpec(
        num_scalar_prefetch=1,
        grid=(B,),
        in_specs=[pl.BlockSpec((S, D), lambda b, idx: (b, 0)),
                  pl.BlockSpec((1, E, D), lambda b, idx: (idx[b], 0, 0)),
                  pl.BlockSpec((1, 1, E), lambda b, idx: (idx[b], 0, 0))],
        out_specs=[pl.BlockSpec((1, 1, S), lambda b, idx: (b, 0, 0)),
                   pl.BlockSpec((1, 1, S), lambda b, idx: (b, 0, 0))],
    )
    gate, gval = pl.pallas_call(
        _gate_body,
        grid_spec=gate_spec,
        out_shape=[jax.ShapeDtypeStruct((B, 1, S), jnp.int32),
                   jax.ShapeDtypeStruct((B, 1, S), jnp.float32)],
    )(idxes, x, gate_W, gate_b.reshape(ND, 1, E))

    gate_flat = gate.reshape(TROWS, 128)
    dst, bex = pl.pallas_call(
        _route_body,
        grid=(1,),
        in_specs=[pl.BlockSpec((TROWS, 128), lambda j: (0, 0))],
        out_specs=[pl.BlockSpec((TROWS, 128), lambda j: (0, 0)),
                   pl.BlockSpec((1, NB), lambda j: (0, 0))],
        out_shape=[jax.ShapeDtypeStruct((TROWS, 128), jnp.int32),
                   jax.ShapeDtypeStruct((1, NB), jnp.int32)],
    )(gate_flat)

    xs = _sc_scatter(x, dst.reshape(T), PAD_T)

    expert_spec = pltpu.PrefetchScalarGridSpec(
        num_scalar_prefetch=1,
        grid=(NB,),
        in_specs=[pl.BlockSpec((TBS, D), lambda i, bx: (i, 0)),
                  pl.BlockSpec((1, INTER, D), lambda i, bx: (bx[i], 0, 0)),
                  pl.BlockSpec((1, 1, INTER), lambda i, bx: (bx[i], 0, 0)),
                  pl.BlockSpec((1, D, INTER), lambda i, bx: (bx[i], 0, 0))],
        out_specs=pl.BlockSpec((TBS, D), lambda i, bx: (i, 0)),
    )
    ys = pl.pallas_call(
        _expert_body,
        grid_spec=expert_spec,
        out_shape=jax.ShapeDtypeStruct((PAD_T, D), jnp.float32),
    )(bex.reshape(NB), xs, exp1_W,
      exp1_b.reshape(E, 1, INTER), exp2_W)

    ye = _sc_gather(ys, dst.reshape(T), T)

    gval_r = gval.reshape(NTB, 1, TB)
    out = pl.pallas_call(
        _combine_body,
        grid=(NTB,),
        in_specs=[pl.BlockSpec((TB, D), lambda i: (i, 0)),
                  pl.BlockSpec((TB, D), lambda i: (i, 0)),
                  pl.BlockSpec((1, D), lambda i: (0, 0)),
                  pl.BlockSpec((1, 1, TB), lambda i: (i, 0, 0)),
                  pl.BlockSpec((TB, D), lambda i: (i, 0)),
                  pl.BlockSpec((1, D), lambda i: (0, 0)),
                  pl.BlockSpec((1, D), lambda i: (0, 0))],
        out_specs=pl.BlockSpec((TB, D), lambda i: (i, 0)),
        out_shape=jax.ShapeDtypeStruct((T, D), jnp.float32),
    )(y_sh, ye, fc2_b.reshape(1, D), gval_r, x,
      fln_g.reshape(1, D), fln_b.reshape(1, D))

    return out.reshape(B, S, D)


# speedup vs baseline: 1.6704x; 1.0007x over previous
import functools

import jax
import jax.numpy as jnp
from jax.experimental import pallas as pl
from jax.experimental.pallas import tpu as pltpu
from jax.experimental.pallas import tpu_sc as plsc

B, S, D, H = 2, 2048, 768, 12
FFN, INTER, E, ND = 3072, 768, 8, 4
DH = D // H
T = B * S
SCALE = DH ** -0.5

TB_QKV = 512
BQ = 2048
TB = 256
NTB = T // TB

TBS = 128
NB = 40
PAD_T = NB * TBS

TROWS = T // 128


def _gelu(x):
    return x * 0.5 * (1.0 + jax.lax.erf(x * (2.0 ** -0.5)))


def _layernorm(t, g, b):
    m = jnp.mean(t, axis=-1, keepdims=True)
    v = jnp.mean((t - m) ** 2, axis=-1, keepdims=True)
    return (t - m) / jnp.sqrt(v + 1e-5) * g + b


def _qkv_body(x_ref, wq_ref, wk_ref, wv_ref, b_ref, q_ref, k_ref, v_ref):
    x = x_ref[...]
    cn = (((1,), (1,)), ((), ()))
    q = (jax.lax.dot_general(x, wq_ref[...], cn,
                             preferred_element_type=jnp.float32)
         + b_ref[:, :D]) * SCALE
    k = (jax.lax.dot_general(x, wk_ref[...], cn,
                             preferred_element_type=jnp.float32)
         + b_ref[:, D:2 * D])
    v = (jax.lax.dot_general(x, wv_ref[...], cn,
                             preferred_element_type=jnp.float32)
         + b_ref[:, 2 * D:])
    q_ref[0] = jnp.stack([q[:, h * DH:(h + 1) * DH] for h in range(H)], 0)
    k_ref[0] = jnp.stack([k[:, h * DH:(h + 1) * DH] for h in range(H)], 0)
    v_ref[0] = jnp.stack([v[:, h * DH:(h + 1) * DH] for h in range(H)], 0)


def _attn_body(q_ref, k_ref, v_ref, o_ref):
    q = q_ref[0, 0]
    k = k_ref[0, 0]
    v = v_ref[0, 0]
    s = jax.lax.dot_general(q, k, (((1,), (1,)), ((), ())),
                            preferred_element_type=jnp.float32)
    m = jnp.max(s, axis=-1, keepdims=True)
    p = jnp.exp(s - m)
    p = p / jnp.sum(p, axis=-1, keepdims=True)
    o_ref[0, 0] = jnp.dot(p, v, preferred_element_type=jnp.float32)


def _oproj_ln_body(o_ref, w_ref, b_ref, hs_ref, g_ref, bb_ref, x_ref):
    o = jnp.concatenate([o_ref[0, h] for h in range(H)], axis=-1)
    t = (jax.lax.dot_general(o, w_ref[...], (((1,), (1,)), ((), ())),
                             preferred_element_type=jnp.float32)
         + b_ref[...] + hs_ref[...])
    x_ref[...] = _layernorm(t, g_ref[...], bb_ref[...])


def _gate_body(idx_ref, x_ref, gw_ref, gb_ref, gate_ref, gval_ref):
    del idx_ref
    x = x_ref[...]
    gw = gw_ref[0]
    logits = jax.lax.dot_general(x, gw, (((1,), (1,)), ((), ())),
                                 preferred_element_type=jnp.float32)
    logits = logits + gb_ref[0]
    m = jnp.max(logits, axis=-1, keepdims=True)
    p = jnp.exp(logits - m)
    sp = jnp.sum(p, axis=-1)
    gate_ref[0, 0] = jnp.argmax(logits, axis=-1).astype(jnp.int32)
    gval_ref[0, 0] = jnp.max(p, axis=-1) / sp


def _route_body(gate_ref, dst_ref, bex_ref):
    g = gate_ref[...]
    l0 = jax.lax.broadcasted_iota(jnp.int32, (128, 128), 0)
    l1 = jax.lax.broadcasted_iota(jnp.int32, (128, 128), 1)
    slt_lane = (l0 < l1).astype(jnp.float32)
    r0 = jax.lax.broadcasted_iota(jnp.int32, (TROWS, TROWS), 0)
    r1 = jax.lax.broadcasted_iota(jnp.int32, (TROWS, TROWS), 1)
    slt_row = (r1 < r0).astype(jnp.float32)

    dst = jnp.zeros((TROWS, 128), jnp.float32)
    bpos = (jax.lax.broadcasted_iota(jnp.int32, (1, NB), 1)
            * TBS).astype(jnp.float32)
    bex = jnp.zeros((1, NB), jnp.float32)
    base = jnp.float32(0.0)
    for e in range(E):
        oh = (g == e).astype(jnp.float32)
        cs = jnp.dot(oh, slt_lane, preferred_element_type=jnp.float32)
        rs = jnp.sum(oh, axis=1, keepdims=True)
        rp = jnp.dot(slt_row, rs, preferred_element_type=jnp.float32)
        dst = dst + oh * (cs + rp + base)
        cnt = jnp.sum(oh)
        pc = jnp.ceil(cnt * (1.0 / TBS)) * TBS
        bex = bex + jnp.float32(e) * ((bpos >= base) &
                                      (bpos < base + pc)).astype(jnp.float32)
        base = base + pc
    dst_ref[...] = dst.astype(jnp.int32)
    bex_ref[...] = bex.astype(jnp.int32)


def _expert_body(bex_ref, xs_ref, e1w_ref, e1b_ref, e2w_ref, ys_ref):
    del bex_ref
    xs = xs_ref[...]
    he = jax.lax.dot_general(xs, e1w_ref[0], (((1,), (1,)), ((), ())),
                             preferred_element_type=jnp.float32)
    he = _gelu(he + e1b_ref[0])
    ys_ref[...] = jax.lax.dot_general(he, e2w_ref[0], (((1,), (1,)), ((), ())),
                                      preferred_element_type=jnp.float32)


def _shared_body(x_ref, w1_ref, b1_ref, w2_ref, o_ref):
    cn = (((1,), (1,)), ((), ()))
    h = jax.lax.dot_general(x_ref[...], w1_ref[...], cn,
                            preferred_element_type=jnp.float32)
    h = _gelu(h + b1_ref[...])
    o_ref[...] = jax.lax.dot_general(h, w2_ref[...], cn,
                                     preferred_element_type=jnp.float32)


def _combine_body(ysh_ref, ye_ref, b2_ref, gval_ref, x_ref, g_ref, bb_ref,
                  o_ref):
    x = x_ref[...]
    y = ysh_ref[...] + ye_ref[...] + b2_ref[...]
    t = y * gval_ref[0, 0][:, None] + x
    o_ref[...] = _layernorm(t, g_ref[...], bb_ref[...])


def _sc_scatter(table, idx, n_rows_out):
    info = plsc.get_sparse_core_info()
    nc, ns = info.num_cores, info.num_subcores
    nw = nc * ns
    n_rows_in = table.shape[0]
    per_w = n_rows_in // nw
    n_ch = 4
    ch = per_w // n_ch
    mesh = plsc.VectorSubcoreMesh(core_axis_name="c", subcore_axis_name="s")

    @functools.partial(
        pl.kernel, mesh=mesh,
        out_type=jax.ShapeDtypeStruct((n_rows_out, D), jnp.float32),
        scratch_types=[pltpu.VMEM((n_ch, ch), jnp.int32)]
        + [pltpu.VMEM((ch, D), jnp.float32) for _ in range(4)]
        + [pltpu.SemaphoreType.DMA, pltpu.SemaphoreType.DMA])
    def sk(table_hbm, idx_hbm, out_hbm, idx_v, b0, b1, b2, b3, lsem, ssem):
        bufs = [b0, b1, b2, b3]
        wid = jax.lax.axis_index("s") * nc + jax.lax.axis_index("c")
        base = wid * per_w
        loads = []
        for c in range(n_ch):
            pltpu.sync_copy(idx_hbm.at[pl.ds(base + c * ch, ch)], idx_v.at[c])
            loads.append(pltpu.async_copy(
                table_hbm.at[pl.ds(base + c * ch, ch)], bufs[c], lsem))
        stores = []
        for c in range(n_ch):
            loads[c].wait()
            stores.append(pltpu.async_copy(
                bufs[c], out_hbm.at[idx_v.at[c]], ssem))
        for cp in stores:
            cp.wait()

    return sk(table, idx)


def _sc_gather(table, idx, n_rows):
    info = plsc.get_sparse_core_info()
    nc, ns = info.num_cores, info.num_subcores
    nw = nc * ns
    per_w = n_rows // nw
    n_ch = 4
    ch = per_w // n_ch
    mesh = plsc.VectorSubcoreMesh(core_axis_name="c", subcore_axis_name="s")

    @functools.partial(
        pl.kernel, mesh=mesh,
        out_type=jax.ShapeDtypeStruct((n_rows, D), jnp.float32),
        scratch_types=[pltpu.VMEM((per_w,), jnp.int32)]
        + [pltpu.VMEM((ch, D), jnp.float32) for _ in range(4)]
        + [pltpu.SemaphoreType.DMA, pltpu.SemaphoreType.DMA])
    def gk(table_hbm, idx_hbm, out_hbm, idx_v, b0, b1, b2, b3, gsem, ssem):
        bufs = [b0, b1, b2, b3]
        wid = jax.lax.axis_index("s") * nc + jax.lax.axis_index("c")
        base = wid * per_w
        pltpu.sync_copy(idx_hbm.at[pl.ds(base, per_w)], idx_v)
        gathers = [pltpu.async_copy(
            table_hbm.at[idx_v.at[pl.ds(c * ch, ch)]], bufs[c], gsem)
            for c in range(n_ch)]
        stores = []
        for c in range(n_ch):
            gathers[c].wait()
            stores.append(pltpu.async_copy(
                bufs[c], out_hbm.at[pl.ds(base + c * ch, ch)], ssem))
        for cp in stores:
            cp.wait()

    return gk(table, idx)


def kernel(hidden_states, idxes, Wq, bq, Wk, bk, Wv, bv, Wo, bo, ln1_g, ln1_b,
           fc1_W, fc1_b, fc2_W, fc2_b, exp1_W, exp1_b, exp2_W, gate_W, gate_b,
           fln_g, fln_b):
    xf = hidden_states.reshape(T, D)
    bqkv = jnp.concatenate([bq, bk, bv]).reshape(1, 3 * D)
    sb = S // TB_QKV

    hspec = pl.BlockSpec((1, H, TB_QKV, DH), lambda i: (i // sb, 0, i % sb, 0))
    q, k, v = pl.pallas_call(
        _qkv_body,
        grid=(T // TB_QKV,),
        in_specs=[pl.BlockSpec((TB_QKV, D), lambda i: (i, 0)),
                  pl.BlockSpec((D, D), lambda i: (0, 0)),
                  pl.BlockSpec((D, D), lambda i: (0, 0)),
                  pl.BlockSpec((D, D), lambda i: (0, 0)),
                  pl.BlockSpec((1, 3 * D), lambda i: (0, 0))],
        out_specs=[hspec, hspec, hspec],
        out_shape=[jax.ShapeDtypeStruct((B, H, S, DH), jnp.float32)] * 3,
    )(xf, Wq, Wk, Wv, bqkv)

    o = pl.pallas_call(
        _attn_body,
        grid=(B, H, S // BQ),
        in_specs=[pl.BlockSpec((1, 1, BQ, DH), lambda b, h, i: (b, h, i, 0)),
                  pl.BlockSpec((1, 1, S, DH), lambda b, h, i: (b, h, 0, 0)),
                  pl.BlockSpec((1, 1, S, DH), lambda b, h, i: (b, h, 0, 0))],
        out_specs=pl.BlockSpec((1, 1, BQ, DH), lambda b, h, i: (b, h, i, 0)),
        out_shape=jax.ShapeDtypeStruct((B, H, S, DH), jnp.float32),
    )(q, k, v)

    stb = S // TB
    x = pl.pallas_call(
        _oproj_ln_body,
        grid=(NTB,),
        in_specs=[pl.BlockSpec((1, H, TB, DH),
                               lambda i: (i // stb, 0, i % stb, 0)),
                  pl.BlockSpec((D, D), lambda i: (0, 0)),
                  pl.BlockSpec((1, D), lambda i: (0, 0)),
                  pl.BlockSpec((TB, D), lambda i: (i, 0)),
                  pl.BlockSpec((1, D), lambda i: (0, 0)),
                  pl.BlockSpec((1, D), lambda i: (0, 0))],
        out_specs=pl.BlockSpec((TB, D), lambda i: (i, 0)),
        out_shape=jax.ShapeDtypeStruct((T, D), jnp.float32),
    )(o, Wo, bo.reshape(1, D), xf,
      ln1_g.reshape(1, D), ln1_b.reshape(1, D))

    y_sh = pl.pallas_call(
        _shared_body,
        grid=(NTB,),
        in_specs=[pl.BlockSpec((TB, D), lambda i: (i, 0)),
                  pl.BlockSpec((FFN, D), lambda i: (0, 0)),
                  pl.BlockSpec((1, FFN), lambda i: (0, 0)),
                  pl.BlockSpec((D, FFN), lambda i: (0, 0))],
        out_specs=pl.BlockSpec((TB, D), lambda i: (i, 0)),
        out_shape=jax.ShapeDtypeStruct((T, D), jnp.float32),
    )(x, fc1_W, fc1_b.reshape(1, FFN), fc2_W)

    gate_spec = pltpu.PrefetchScalarGridSpec(
        num_scalar_prefetch=1,
        grid=(B,),
        in_specs=[pl.BlockSpec((S, D), lambda b, idx: (b, 0)),
                  pl.BlockSpec((1, E, D), lambda b, idx: (idx[b], 0, 0)),
                  pl.BlockSpec((1, 1, E), lambda b, idx: (idx[b], 0, 0))],
        out_specs=[pl.BlockSpec((1, 1, S), lambda b, idx: (b, 0, 0)),
                   pl.BlockSpec((1, 1, S), lambda b, idx: (b, 0, 0))],
    )
    gate, gval = pl.pallas_call(
        _gate_body,
        grid_spec=gate_spec,
        out_shape=[jax.ShapeDtypeStruct((B, 1, S), jnp.int32),
                   jax.ShapeDtypeStruct((B, 1, S), jnp.float32)],
    )(idxes, x, gate_W, gate_b.reshape(ND, 1, E))

    gate_flat = gate.reshape(TROWS, 128)
    dst, bex = pl.pallas_call(
        _route_body,
        grid=(1,),
        in_specs=[pl.BlockSpec((TROWS, 128), lambda j: (0, 0))],
        out_specs=[pl.BlockSpec((TROWS, 128), lambda j: (0, 0)),
                   pl.BlockSpec((1, NB), lambda j: (0, 0))],
        out_shape=[jax.ShapeDtypeStruct((TROWS, 128), jnp.int32),
                   jax.ShapeDtypeStruct((1, NB), jnp.int32)],
    )(gate_flat)

    xs = _sc_scatter(x, dst.reshape(T), PAD_T)

    expert_spec = pltpu.PrefetchScalarGridSpec(
        num_scalar_prefetch=1,
        grid=(NB,),
        in_specs=[pl.BlockSpec((TBS, D), lambda i, bx: (i, 0)),
                  pl.BlockSpec((1, INTER, D), lambda i, bx: (bx[i], 0, 0)),
                  pl.BlockSpec((1, 1, INTER), lambda i, bx: (bx[i], 0, 0)),
                  pl.BlockSpec((1, D, INTER), lambda i, bx: (bx[i], 0, 0))],
        out_specs=pl.BlockSpec((TBS, D), lambda i, bx: (i, 0)),
    )
    ys = pl.pallas_call(
        _expert_body,
        grid_spec=expert_spec,
        out_shape=jax.ShapeDtypeStruct((PAD_T, D), jnp.float32),
    )(bex.reshape(NB), xs, exp1_W,
      exp1_b.reshape(E, 1, INTER), exp2_W)

    ye = _sc_gather(ys, dst.reshape(T), T)

    gval_r = gval.reshape(NTB, 1, TB)
    out = pl.pallas_call(
        _combine_body,
        grid=(NTB,),
        in_specs=[pl.BlockSpec((TB, D), lambda i: (i, 0)),
                  pl.BlockSpec((TB, D), lambda i: (i, 0)),
                  pl.BlockSpec((1, D), lambda i: (0, 0)),
                  pl.BlockSpec((1, 1, TB), lambda i: (i, 0, 0)),
                  pl.BlockSpec((TB, D), lambda i: (i, 0)),
                  pl.BlockSpec((1, D), lambda i: (0, 0)),
                  pl.BlockSpec((1, D), lambda i: (0, 0))],
        out_specs=pl.BlockSpec((TB, D), lambda i: (i, 0)),
        out_shape=jax.ShapeDtypeStruct((T, D), jnp.float32),
    )(y_sh, ye, fc2_b.reshape(1, D), gval_r, x,
      fln_g.reshape(1, D), fln_b.reshape(1, D))

    return out.reshape(B, S, D)
